# Initial kernel scaffold; baseline (speedup 1.0000x reference)
import functools

import jax
import jax.numpy as jnp
from jax import lax
from jax.experimental import pallas as pl
from jax.experimental.pallas import tpu as pltpu
from jax.experimental.pallas import tpu_sc as plsc

N = 10000
E = 320000
D = 128
H = 128
EPS = 1e-5

NC = 2
NS = 16
HH = H // NC
K = 80
EPT = E // NS
NCHUNK = EPT // K
RPT = N // NS
ZR = 125
DW = 16

ROWBLK = 1000


def _zero_fill(ref, rows, width):
  z = jnp.zeros((16,), jnp.float32)
  def body(i, _):
    for j in range(width // 16):
      ref[i, pl.ds(j * 16, 16)] = z
    return 0
  lax.fori_loop(0, rows, body, 0)


def _fill_ones(ref, rows):
  o = jnp.ones((16,), jnp.float32)
  def body(i, _):
    ref[i, :] = o
    return 0
  lax.fori_loop(0, rows, body, 0)


def _sc_agg_body(with_deg, *refs):
  if with_deg:
    (h_hbm, src_hbm, dst_hbm, agg_out, deg_out,
     h_sh, agg_sh, deg_sh, src_v, dst_v, rows_v, zbuf, zdeg, ones_v,
     gsem) = refs
  else:
    (h_hbm, src_hbm, dst_hbm, agg_out,
     h_sh, agg_sh, src_v, dst_v, rows_v, zbuf,
     gsem) = refs

  cid = lax.axis_index("c")
  sid = lax.axis_index("s")
  row0 = sid * RPT

  pltpu.sync_copy(h_hbm.at[cid, pl.ds(row0, RPT)], h_sh.at[pl.ds(row0, RPT)])

  _zero_fill(zbuf, ZR, HH)
  for i in range(RPT // ZR):
    pltpu.sync_copy(zbuf, agg_sh.at[pl.ds(row0 + i * ZR, ZR)])
  if with_deg:
    @pl.when(cid == 0)
    def _():
      _zero_fill(zdeg, ZR, DW)
      _fill_ones(ones_v, K)
      for i in range(RPT // ZR):
        pltpu.sync_copy(zdeg, deg_sh.at[pl.ds(row0 + i * ZR, ZR)])

  pltpu.sync_copy(src_hbm.at[sid], src_v)
  pltpu.sync_copy(dst_hbm.at[sid], dst_v)

  plsc.subcore_barrier()

  def chunk(j, _):
    pltpu.async_copy(h_sh.at[src_v.at[j]], rows_v, gsem).wait()
    pltpu.sync_copy(rows_v, agg_sh.at[dst_v.at[j]], add=True)
    if with_deg:
      @pl.when(cid == 0)
      def _():
        pltpu.sync_copy(ones_v, deg_sh.at[dst_v.at[j]], add=True)
    return 0
  lax.fori_loop(0, NCHUNK, chunk, 0)

  plsc.subcore_barrier()

  pltpu.sync_copy(agg_sh.at[pl.ds(row0, RPT)], agg_out.at[cid, pl.ds(row0, RPT)])
  if with_deg:
    @pl.when(cid == 0)
    def _():
      pltpu.sync_copy(deg_sh.at[pl.ds(row0, RPT)], deg_out.at[pl.ds(row0, RPT)])


def _make_sc_agg(with_deg):
  mesh = plsc.VectorSubcoreMesh(core_axis_name="c", subcore_axis_name="s")
  out_type = [jax.ShapeDtypeStruct((NC, N, HH), jnp.float32)]
  scratch = [
      pltpu.VMEM_SHARED((N, HH), jnp.float32),
      pltpu.VMEM_SHARED((N, HH), jnp.float32),
  ]
  if with_deg:
    out_type.append(jax.ShapeDtypeStruct((N, DW), jnp.float32))
    scratch.append(pltpu.VMEM_SHARED((N, DW), jnp.float32))
  scratch += [
      pltpu.VMEM((NCHUNK, K), jnp.int32),
      pltpu.VMEM((NCHUNK, K), jnp.int32),
      pltpu.VMEM((K, HH), jnp.float32),
      pltpu.VMEM((ZR, HH), jnp.float32),
  ]
  if with_deg:
    scratch += [
        pltpu.VMEM((ZR, DW), jnp.float32),
        pltpu.VMEM((K, DW), jnp.float32),
    ]
  scratch.append(pltpu.SemaphoreType.DMA)
  return pl.kernel(
      functools.partial(_sc_agg_body, with_deg),
      out_type=tuple(out_type),
      mesh=mesh,
      scratch_types=tuple(scratch),
  )


def _pre_body(x_ref, w_ref, b_ref, hf_ref, hs_ref):
  h = jnp.dot(x_ref[...], w_ref[...], preferred_element_type=jnp.float32)
  h = jnp.maximum(h + b_ref[...], 0.0)
  hf_ref[...] = h
  hs_ref[0] = h[:, :HH]
  hs_ref[1] = h[:, HH:]


def _conv_body(head, *refs):
  if head:
    (agg_ref, deg_ref, h_ref, llw_ref, llb_ref, lrw_ref, g_ref, b_ref,
     hw_ref, hb_ref, hf_ref, y_ref) = refs
  else:
    (agg_ref, deg_ref, h_ref, llw_ref, llb_ref, lrw_ref, g_ref, b_ref,
     hf_ref, hs_ref) = refs
  inv = 1.0 / jnp.maximum(deg_ref[...][:, 0:1], 1.0)
  llw = llw_ref[...]
  z = jnp.dot(agg_ref[0] * inv, llw[:HH, :], preferred_element_type=jnp.float32)
  z = z + jnp.dot(agg_ref[1] * inv, llw[HH:, :], preferred_element_type=jnp.float32)
  z = z + jnp.dot(h_ref[...], lrw_ref[...], preferred_element_type=jnp.float32)
  z = z + llb_ref[...]
  bn_scale = g_ref[...] * (1.0 / jnp.sqrt(1.0 + EPS))
  h = jnp.maximum(z * bn_scale + b_ref[...], 0.0)
  hf_ref[...] = h
  if head:
    y_ref[...] = jnp.dot(h, hw_ref[...], preferred_element_type=jnp.float32) + hb_ref[...]
  else:
    hs_ref[0] = h[:, :HH]
    hs_ref[1] = h[:, HH:]


_row_spec = pl.BlockSpec((ROWBLK, H), lambda i: (i, 0))
_split_spec = pl.BlockSpec((NC, ROWBLK, HH), lambda i: (0, i, 0))
_full_spec = pl.BlockSpec((H, H), lambda i: (0, 0))
_vec_spec = pl.BlockSpec((1, H), lambda i: (0, 0))


def _tc_pre(x, w, b):
  return pl.pallas_call(
      _pre_body,
      grid=(N // ROWBLK,),
      in_specs=[_row_spec, _full_spec, _vec_spec],
      out_specs=[_row_spec, _split_spec],
      out_shape=[
          jax.ShapeDtypeStruct((N, H), jnp.float32),
          jax.ShapeDtypeStruct((NC, N, HH), jnp.float32),
      ],
  )(x, w, b)


def _tc_conv(agg, deg, h, llw, llb, lrw, g, b):
  return pl.pallas_call(
      functools.partial(_conv_body, False),
      grid=(N // ROWBLK,),
      in_specs=[
          _split_spec,
          pl.BlockSpec((ROWBLK, DW), lambda i: (i, 0)),
          _row_spec, _full_spec, _vec_spec, _full_spec, _vec_spec, _vec_spec,
      ],
      out_specs=[_row_spec, _split_spec],
      out_shape=[
          jax.ShapeDtypeStruct((N, H), jnp.float32),
          jax.ShapeDtypeStruct((NC, N, HH), jnp.float32),
      ],
  )(agg, deg, h, llw, llb, lrw, g, b)


def _tc_conv_head(agg, deg, h, llw, llb, lrw, g, b, hw, hb):
  return pl.pallas_call(
      functools.partial(_conv_body, True),
      grid=(N // ROWBLK,),
      in_specs=[
          _split_spec,
          pl.BlockSpec((ROWBLK, DW), lambda i: (i, 0)),
          _row_spec, _full_spec, _vec_spec, _full_spec, _vec_spec, _vec_spec,
          pl.BlockSpec((H, 1), lambda i: (0, 0)),
          pl.BlockSpec((1, 1), lambda i: (0, 0)),
      ],
      out_specs=[_row_spec, pl.BlockSpec((ROWBLK, 1), lambda i: (i, 0))],
      out_shape=[
          jax.ShapeDtypeStruct((N, H), jnp.float32),
          jax.ShapeDtypeStruct((N, 1), jnp.float32),
      ],
  )(agg, deg, h, llw, llb, lrw, g, b, hw, hb)


def kernel(x, edge_index, pre_w, pre_b, c0_ll_w, c0_ll_b, c0_lr_w,
           c1_ll_w, c1_ll_b, c1_lr_w, n0_g, n0_b, n1_g, n1_b,
           head_w, head_b):
  src = edge_index[0].reshape(NS, NCHUNK, K)
  dst = edge_index[1].reshape(NS, NCHUNK, K)

  h0, h0s = _tc_pre(x, pre_w, pre_b.reshape(1, H))

  agg0, deg = _make_sc_agg(True)(h0s, src, dst)
  h1, h1s = _tc_conv(agg0, deg, h0, c0_ll_w, c0_ll_b.reshape(1, H),
                     c0_lr_w, n0_g.reshape(1, H), n0_b.reshape(1, H))

  (agg1,) = _make_sc_agg(False)(h1s, src, dst)
  h2, y = _tc_conv_head(agg1, deg, h1, c1_ll_w, c1_ll_b.reshape(1, H),
                        c1_lr_w, n1_g.reshape(1, H), n1_b.reshape(1, H),
                        head_w, head_b.reshape(1, 1))
  return (y[:, 0], h2)


# speedup vs baseline: 5.9399x; 5.9399x over previous
import functools

import jax
import jax.numpy as jnp
from jax import lax
from jax.experimental import pallas as pl
from jax.experimental.pallas import tpu as pltpu
from jax.experimental.pallas import tpu_sc as plsc

N = 10000
E = 320000
D = 128
H = 128
EPS = 1e-5

NC = 2
NS = 16
HH = H // NC
NP = 10240
K = 128
NCHUNK = 160
EP = NS * NCHUNK * K
PAD_DST = 10200
RPT = NP // NS
ZR = 64
DW = 16
IDXB = 16

ROWBLK = 1000


def _zero_fill(ref, rows, width):
  z = jnp.zeros((16,), jnp.float32)
  def body(i, _):
    for j in range(width // 16):
      ref[i, pl.ds(j * 16, 16)] = z
    return 0
  lax.fori_loop(0, rows, body, 0)


def _fill_ones(ref, rows):
  o = jnp.ones((16,), jnp.float32)
  def body(i, _):
    ref[i, :] = o
    return 0
  lax.fori_loop(0, rows, body, 0)


def _sc_agg_body(with_deg, *refs):
  if with_deg:
    (h_hbm, src_hbm, dst_hbm, agg_out, deg_out,
     h_sh, agg_sh, deg_sh, src_v, dst_v, rows_v, zbuf, zdeg, ones_v,
     gsem) = refs
  else:
    (h_hbm, src_hbm, dst_hbm, agg_out,
     h_sh, agg_sh, src_v, dst_v, rows_v, zbuf,
     gsem) = refs

  cid = lax.axis_index("c")
  sid = lax.axis_index("s")
  row0 = sid * RPT

  pltpu.sync_copy(h_hbm.at[cid, pl.ds(row0, RPT)], h_sh.at[pl.ds(row0, RPT)])

  _zero_fill(zbuf, ZR, HH)
  for i in range(RPT // ZR):
    pltpu.sync_copy(zbuf, agg_sh.at[pl.ds(row0 + i * ZR, ZR)])
  if with_deg:
    @pl.when(cid == 0)
    def _():
      _zero_fill(zdeg, ZR, DW)
      _fill_ones(ones_v, K)
      for i in range(RPT // ZR):
        pltpu.sync_copy(zdeg, deg_sh.at[pl.ds(row0 + i * ZR, ZR)])

  plsc.subcore_barrier()

  def block(ib, _):
    pltpu.sync_copy(src_hbm.at[sid, pl.ds(ib * IDXB, IDXB)], src_v)
    pltpu.sync_copy(dst_hbm.at[sid, pl.ds(ib * IDXB, IDXB)], dst_v)
    def chunk(q, _):
      pltpu.async_copy(h_sh.at[src_v.at[q]], rows_v, gsem).wait()
      pltpu.sync_copy(rows_v, agg_sh.at[dst_v.at[q]], add=True)
      if with_deg:
        @pl.when(cid == 0)
        def _():
          pltpu.sync_copy(ones_v, deg_sh.at[dst_v.at[q]], add=True)
      return 0
    lax.fori_loop(0, IDXB, chunk, 0)
    return 0
  lax.fori_loop(0, NCHUNK // IDXB, block, 0)

  plsc.subcore_barrier()

  pltpu.sync_copy(agg_sh.at[pl.ds(row0, RPT)], agg_out.at[cid, pl.ds(row0, RPT)])
  if with_deg:
    @pl.when(cid == 0)
    def _():
      pltpu.sync_copy(deg_sh.at[pl.ds(row0, RPT)], deg_out.at[pl.ds(row0, RPT)])


def _make_sc_agg(with_deg):
  mesh = plsc.VectorSubcoreMesh(core_axis_name="c", subcore_axis_name="s")
  out_type = [jax.ShapeDtypeStruct((NC, NP, HH), jnp.float32)]
  scratch = [
      pltpu.VMEM_SHARED((NP, HH), jnp.float32),
      pltpu.VMEM_SHARED((NP, HH), jnp.float32),
  ]
  if with_deg:
    out_type.append(jax.ShapeDtypeStruct((NP, DW), jnp.float32))
    scratch.append(pltpu.VMEM_SHARED((NP, DW), jnp.float32))
  scratch += [
      pltpu.VMEM((IDXB, K), jnp.int32),
      pltpu.VMEM((IDXB, K), jnp.int32),
      pltpu.VMEM((K, HH), jnp.float32),
      pltpu.VMEM((ZR, HH), jnp.float32),
  ]
  if with_deg:
    scratch += [
        pltpu.VMEM((ZR, DW), jnp.float32),
        pltpu.VMEM((K, DW), jnp.float32),
    ]
  scratch.append(pltpu.SemaphoreType.DMA)
  return pl.kernel(
      functools.partial(_sc_agg_body, with_deg),
      out_type=tuple(out_type),
      mesh=mesh,
      scratch_types=tuple(scratch),
      compiler_params=pltpu.CompilerParams(use_tc_tiling_on_sc=False),
  )


def _pre_body(x_ref, w_ref, b_ref, hf_ref, hs_ref):
  h = jnp.dot(x_ref[...], w_ref[...], preferred_element_type=jnp.float32)
  h = jnp.maximum(h + b_ref[...], 0.0)
  hf_ref[...] = h
  hs_ref[0] = h[:, :HH]
  hs_ref[1] = h[:, HH:]


def _conv_body(head, *refs):
  if head:
    (agg_ref, deg_ref, h_ref, llw_ref, llb_ref, lrw_ref, g_ref, b_ref,
     hw_ref, hb_ref, hf_ref, y_ref) = refs
  else:
    (agg_ref, deg_ref, h_ref, llw_ref, llb_ref, lrw_ref, g_ref, b_ref,
     hf_ref, hs_ref) = refs
  inv = 1.0 / jnp.maximum(deg_ref[...][:, 0:1], 1.0)
  llw = llw_ref[...]
  z = jnp.dot(agg_ref[0] * inv, llw[:HH, :], preferred_element_type=jnp.float32)
  z = z + jnp.dot(agg_ref[1] * inv, llw[HH:, :], preferred_element_type=jnp.float32)
  z = z + jnp.dot(h_ref[...], lrw_ref[...], preferred_element_type=jnp.float32)
  z = z + llb_ref[...]
  bn_scale = g_ref[...] * (1.0 / jnp.sqrt(1.0 + EPS))
  h = jnp.maximum(z * bn_scale + b_ref[...], 0.0)
  hf_ref[...] = h
  if head:
    y_ref[...] = jnp.dot(h, hw_ref[...], preferred_element_type=jnp.float32) + hb_ref[...]
  else:
    hs_ref[0] = h[:, :HH]
    hs_ref[1] = h[:, HH:]


_row_spec = pl.BlockSpec((ROWBLK, H), lambda i: (i, 0))
_split_spec = pl.BlockSpec((NC, ROWBLK, HH), lambda i: (0, i, 0))
_full_spec = pl.BlockSpec((H, H), lambda i: (0, 0))
_vec_spec = pl.BlockSpec((1, H), lambda i: (0, 0))


def _tc_pre(x, w, b):
  return pl.pallas_call(
      _pre_body,
      grid=(N // ROWBLK,),
      in_specs=[_row_spec, _full_spec, _vec_spec],
      out_specs=[_row_spec, _split_spec],
      out_shape=[
          jax.ShapeDtypeStruct((N, H), jnp.float32),
          jax.ShapeDtypeStruct((NC, NP, HH), jnp.float32),
      ],
  )(x, w, b)


def _tc_conv(agg, deg, h, llw, llb, lrw, g, b):
  return pl.pallas_call(
      functools.partial(_conv_body, False),
      grid=(N // ROWBLK,),
      in_specs=[
          _split_spec,
          pl.BlockSpec((ROWBLK, DW), lambda i: (i, 0)),
          _row_spec, _full_spec, _vec_spec, _full_spec, _vec_spec, _vec_spec,
      ],
      out_specs=[_row_spec, _split_spec],
      out_shape=[
          jax.ShapeDtypeStruct((N, H), jnp.float32),
          jax.ShapeDtypeStruct((NC, NP, HH), jnp.float32),
      ],
  )(agg, deg, h, llw, llb, lrw, g, b)


def _tc_conv_head(agg, deg, h, llw, llb, lrw, g, b, hw, hb):
  return pl.pallas_call(
      functools.partial(_conv_body, True),
      grid=(N // ROWBLK,),
      in_specs=[
          _split_spec,
          pl.BlockSpec((ROWBLK, DW), lambda i: (i, 0)),
          _row_spec, _full_spec, _vec_spec, _full_spec, _vec_spec, _vec_spec,
          pl.BlockSpec((H, 1), lambda i: (0, 0)),
          pl.BlockSpec((1, 1), lambda i: (0, 0)),
      ],
      out_specs=[_row_spec, pl.BlockSpec((ROWBLK, 1), lambda i: (i, 0))],
      out_shape=[
          jax.ShapeDtypeStruct((N, H), jnp.float32),
          jax.ShapeDtypeStruct((N, 1), jnp.float32),
      ],
  )(agg, deg, h, llw, llb, lrw, g, b, hw, hb)


def kernel(x, edge_index, pre_w, pre_b, c0_ll_w, c0_ll_b, c0_lr_w,
           c1_ll_w, c1_ll_b, c1_lr_w, n0_g, n0_b, n1_g, n1_b,
           head_w, head_b):
  pad = EP - E
  src = jnp.concatenate(
      [edge_index[0], jnp.zeros((pad,), jnp.int32)]).reshape(NS, NCHUNK, K)
  dst = jnp.concatenate(
      [edge_index[1], jnp.full((pad,), PAD_DST, jnp.int32)]).reshape(NS, NCHUNK, K)

  h0, h0s = _tc_pre(x, pre_w, pre_b.reshape(1, H))

  agg0, deg = _make_sc_agg(True)(h0s, src, dst)
  h1, h1s = _tc_conv(agg0, deg, h0, c0_ll_w, c0_ll_b.reshape(1, H),
                     c0_lr_w, n0_g.reshape(1, H), n0_b.reshape(1, H))

  (agg1,) = _make_sc_agg(False)(h1s, src, dst)
  h2, y = _tc_conv_head(agg1, deg, h1, c1_ll_w, c1_ll_b.reshape(1, H),
                        c1_lr_w, n1_g.reshape(1, H), n1_b.reshape(1, H),
                        head_w, head_b.reshape(1, 1))
  return (y[:, 0], h2)


# speedup vs baseline: 6.9572x; 1.1713x over previous
import functools

import jax
import jax.numpy as jnp
from jax import lax
from jax.experimental import pallas as pl
from jax.experimental.pallas import tpu as pltpu
from jax.experimental.pallas import tpu_sc as plsc

N = 10000
E = 320000
D = 128
H = 128
EPS = 1e-5

NC = 2
NS = 16
HH = H // NC
NP = 10240
K = 128
NCHUNK = 160
EP = NS * NCHUNK * K
PAD_DST = 10200
RPT = NP // NS
ZR = 64
DW = 16
IDXB = 16

ROWBLK = 1000


def _zero_fill(ref, rows, width):
  z = jnp.zeros((16,), jnp.float32)
  def body(i, _):
    for j in range(width // 16):
      ref[i, pl.ds(j * 16, 16)] = z
    return 0
  lax.fori_loop(0, rows, body, 0)


def _fill_ones(ref, rows):
  o = jnp.ones((16,), jnp.float32)
  def body(i, _):
    ref[i, :] = o
    return 0
  lax.fori_loop(0, rows, body, 0)


def _sc_agg_body(with_deg, *refs):
  if with_deg:
    (h_hbm, src_hbm, dst_hbm, agg_out, deg_out,
     h_sh, agg_sh, deg_sh, src_v, dst_v, rows_v, zbuf, zdeg, ones_v,
     gsem) = refs
  else:
    (h_hbm, src_hbm, dst_hbm, agg_out,
     h_sh, agg_sh, src_v, dst_v, rows_v, zbuf,
     gsem) = refs

  cid = lax.axis_index("c")
  sid = lax.axis_index("s")
  row0 = sid * RPT

  pltpu.sync_copy(h_hbm.at[cid, pl.ds(row0, RPT)], h_sh.at[pl.ds(row0, RPT)])

  _zero_fill(zbuf, ZR, HH)
  for i in range(RPT // ZR):
    pltpu.sync_copy(zbuf, agg_sh.at[pl.ds(row0 + i * ZR, ZR)])
  if with_deg:
    @pl.when(cid == 0)
    def _():
      _zero_fill(zdeg, ZR, DW)
      _fill_ones(ones_v, K)
      for i in range(RPT // ZR):
        pltpu.sync_copy(zdeg, deg_sh.at[pl.ds(row0 + i * ZR, ZR)])

  plsc.subcore_barrier()

  def gather(q, b):
    return pltpu.make_async_copy(h_sh.at[src_v.at[q]], rows_v.at[b], gsem.at[b])

  def scatter(q, b):
    pltpu.sync_copy(rows_v.at[b], agg_sh.at[dst_v.at[q]], add=True)
    if with_deg:
      @pl.when(cid == 0)
      def _():
        pltpu.sync_copy(ones_v, deg_sh.at[dst_v.at[q]], add=True)

  def block(ib, _):
    pltpu.sync_copy(src_hbm.at[sid, pl.ds(ib * IDXB, IDXB)], src_v)
    pltpu.sync_copy(dst_hbm.at[sid, pl.ds(ib * IDXB, IDXB)], dst_v)
    gather(0, 0).start()
    def pair(t, _):
      q0 = 2 * t
      gather(q0 + 1, 1).start()
      gather(q0, 0).wait()
      scatter(q0, 0)
      @pl.when(t < IDXB // 2 - 1)
      def _():
        gather(q0 + 2, 0).start()
      gather(q0 + 1, 1).wait()
      scatter(q0 + 1, 1)
      return 0
    lax.fori_loop(0, IDXB // 2, pair, 0)
    return 0
  lax.fori_loop(0, NCHUNK // IDXB, block, 0)

  plsc.subcore_barrier()

  pltpu.sync_copy(agg_sh.at[pl.ds(row0, RPT)], agg_out.at[cid, pl.ds(row0, RPT)])
  if with_deg:
    @pl.when(cid == 0)
    def _():
      pltpu.sync_copy(deg_sh.at[pl.ds(row0, RPT)], deg_out.at[pl.ds(row0, RPT)])


def _make_sc_agg(with_deg):
  mesh = plsc.VectorSubcoreMesh(core_axis_name="c", subcore_axis_name="s")
  out_type = [jax.ShapeDtypeStruct((NC, NP, HH), jnp.float32)]
  scratch = [
      pltpu.VMEM_SHARED((NP, HH), jnp.float32),
      pltpu.VMEM_SHARED((NP, HH), jnp.float32),
  ]
  if with_deg:
    out_type.append(jax.ShapeDtypeStruct((NP, DW), jnp.float32))
    scratch.append(pltpu.VMEM_SHARED((NP, DW), jnp.float32))
  scratch += [
      pltpu.VMEM((IDXB, K), jnp.int32),
      pltpu.VMEM((IDXB, K), jnp.int32),
      pltpu.VMEM((2, K, HH), jnp.float32),
      pltpu.VMEM((ZR, HH), jnp.float32),
  ]
  if with_deg:
    scratch += [
        pltpu.VMEM((ZR, DW), jnp.float32),
        pltpu.VMEM((K, DW), jnp.float32),
    ]
  scratch.append(pltpu.SemaphoreType.DMA((2,)))
  return pl.kernel(
      functools.partial(_sc_agg_body, with_deg),
      out_type=tuple(out_type),
      mesh=mesh,
      scratch_types=tuple(scratch),
      compiler_params=pltpu.CompilerParams(use_tc_tiling_on_sc=False),
  )


def _pre_body(x_ref, w_ref, b_ref, hf_ref, hs_ref):
  h = jnp.dot(x_ref[...], w_ref[...], preferred_element_type=jnp.float32)
  h = jnp.maximum(h + b_ref[...], 0.0)
  hf_ref[...] = h
  hs_ref[0] = h[:, :HH]
  hs_ref[1] = h[:, HH:]


def _conv_body(head, *refs):
  if head:
    (agg_ref, deg_ref, h_ref, llw_ref, llb_ref, lrw_ref, g_ref, b_ref,
     hw_ref, hb_ref, hf_ref, y_ref) = refs
  else:
    (agg_ref, deg_ref, h_ref, llw_ref, llb_ref, lrw_ref, g_ref, b_ref,
     hf_ref, hs_ref) = refs
  inv = 1.0 / jnp.maximum(deg_ref[...][:, 0:1], 1.0)
  llw = llw_ref[...]
  z = jnp.dot(agg_ref[0] * inv, llw[:HH, :], preferred_element_type=jnp.float32)
  z = z + jnp.dot(agg_ref[1] * inv, llw[HH:, :], preferred_element_type=jnp.float32)
  z = z + jnp.dot(h_ref[...], lrw_ref[...], preferred_element_type=jnp.float32)
  z = z + llb_ref[...]
  bn_scale = g_ref[...] * (1.0 / jnp.sqrt(1.0 + EPS))
  h = jnp.maximum(z * bn_scale + b_ref[...], 0.0)
  hf_ref[...] = h
  if head:
    y_ref[...] = jnp.dot(h, hw_ref[...], preferred_element_type=jnp.float32) + hb_ref[...]
  else:
    hs_ref[0] = h[:, :HH]
    hs_ref[1] = h[:, HH:]


_row_spec = pl.BlockSpec((ROWBLK, H), lambda i: (i, 0))
_split_spec = pl.BlockSpec((NC, ROWBLK, HH), lambda i: (0, i, 0))
_full_spec = pl.BlockSpec((H, H), lambda i: (0, 0))
_vec_spec = pl.BlockSpec((1, H), lambda i: (0, 0))


def _tc_pre(x, w, b):
  return pl.pallas_call(
      _pre_body,
      grid=(N // ROWBLK,),
      in_specs=[_row_spec, _full_spec, _vec_spec],
      out_specs=[_row_spec, _split_spec],
      out_shape=[
          jax.ShapeDtypeStruct((N, H), jnp.float32),
          jax.ShapeDtypeStruct((NC, NP, HH), jnp.float32),
      ],
  )(x, w, b)


def _tc_conv(agg, deg, h, llw, llb, lrw, g, b):
  return pl.pallas_call(
      functools.partial(_conv_body, False),
      grid=(N // ROWBLK,),
      in_specs=[
          _split_spec,
          pl.BlockSpec((ROWBLK, DW), lambda i: (i, 0)),
          _row_spec, _full_spec, _vec_spec, _full_spec, _vec_spec, _vec_spec,
      ],
      out_specs=[_row_spec, _split_spec],
      out_shape=[
          jax.ShapeDtypeStruct((N, H), jnp.float32),
          jax.ShapeDtypeStruct((NC, NP, HH), jnp.float32),
      ],
  )(agg, deg, h, llw, llb, lrw, g, b)


def _tc_conv_head(agg, deg, h, llw, llb, lrw, g, b, hw, hb):
  return pl.pallas_call(
      functools.partial(_conv_body, True),
      grid=(N // ROWBLK,),
      in_specs=[
          _split_spec,
          pl.BlockSpec((ROWBLK, DW), lambda i: (i, 0)),
          _row_spec, _full_spec, _vec_spec, _full_spec, _vec_spec, _vec_spec,
          pl.BlockSpec((H, 1), lambda i: (0, 0)),
          pl.BlockSpec((1, 1), lambda i: (0, 0)),
      ],
      out_specs=[_row_spec, pl.BlockSpec((ROWBLK, 1), lambda i: (i, 0))],
      out_shape=[
          jax.ShapeDtypeStruct((N, H), jnp.float32),
          jax.ShapeDtypeStruct((N, 1), jnp.float32),
      ],
  )(agg, deg, h, llw, llb, lrw, g, b, hw, hb)


def kernel(x, edge_index, pre_w, pre_b, c0_ll_w, c0_ll_b, c0_lr_w,
           c1_ll_w, c1_ll_b, c1_lr_w, n0_g, n0_b, n1_g, n1_b,
           head_w, head_b):
  pad = EP - E
  src = jnp.concatenate(
      [edge_index[0], jnp.zeros((pad,), jnp.int32)]).reshape(NS, NCHUNK, K)
  dst = jnp.concatenate(
      [edge_index[1], jnp.full((pad,), PAD_DST, jnp.int32)]).reshape(NS, NCHUNK, K)

  h0, h0s = _tc_pre(x, pre_w, pre_b.reshape(1, H))

  agg0, deg = _make_sc_agg(True)(h0s, src, dst)
  h1, h1s = _tc_conv(agg0, deg, h0, c0_ll_w, c0_ll_b.reshape(1, H),
                     c0_lr_w, n0_g.reshape(1, H), n0_b.reshape(1, H))

  (agg1,) = _make_sc_agg(False)(h1s, src, dst)
  h2, y = _tc_conv_head(agg1, deg, h1, c1_ll_w, c1_ll_b.reshape(1, H),
                        c1_lr_w, n1_g.reshape(1, H), n1_b.reshape(1, H),
                        head_w, head_b.reshape(1, 1))
  return (y[:, 0], h2)


# speedup vs baseline: 7.9251x; 1.1391x over previous
import functools

import jax
import jax.numpy as jnp
from jax import lax
from jax.experimental import pallas as pl
from jax.experimental.pallas import tpu as pltpu
from jax.experimental.pallas import tpu_sc as plsc

N = 10000
E = 320000
D = 128
H = 128
EPS = 1e-5

NC = 2
NS = 16
HH = H // NC
NP = 10240
K = 128
NCHUNK = 160
EP = NS * NCHUNK * K
PAD_DST = 10200
RPT = NP // NS
ZR = 64
DW = 16
IDXB = 16

ROWBLK = 1000


def _zero_fill(ref, rows, width):
  z = jnp.zeros((16,), jnp.float32)
  def body(i, _):
    for j in range(width // 16):
      ref[i, pl.ds(j * 16, 16)] = z
    return 0
  lax.fori_loop(0, rows, body, 0)


def _fill_ones(ref, rows):
  o = jnp.ones((16,), jnp.float32)
  def body(i, _):
    ref[i, :] = o
    return 0
  lax.fori_loop(0, rows, body, 0)


def _sc_agg_body(with_deg, *refs):
  if with_deg:
    (h_hbm, src_hbm, dst_hbm, agg_out, deg_out,
     h_sh, agg_sh, deg_sh, src_v, dst_v, rows_v, zbuf, zdeg, ones_v,
     gsem, ssem, dsem) = refs
  else:
    (h_hbm, src_hbm, dst_hbm, agg_out,
     h_sh, agg_sh, src_v, dst_v, rows_v, zbuf,
     gsem, ssem) = refs

  cid = lax.axis_index("c")
  sid = lax.axis_index("s")
  row0 = sid * RPT

  pltpu.sync_copy(h_hbm.at[cid, pl.ds(row0, RPT)], h_sh.at[pl.ds(row0, RPT)])

  _zero_fill(zbuf, ZR, HH)
  for i in range(RPT // ZR):
    pltpu.sync_copy(zbuf, agg_sh.at[pl.ds(row0 + i * ZR, ZR)])
  if with_deg:
    _zero_fill(zdeg, ZR, DW)
    _fill_ones(ones_v, K)
    for i in range(RPT // ZR):
      pltpu.sync_copy(zdeg, deg_sh.at[pl.ds(row0 + i * ZR, ZR)])

  plsc.subcore_barrier()

  NB = 3

  def gather_start(q):
    b = q % NB
    pltpu.async_copy(h_sh.at[src_v.at[q]], rows_v.at[b], gsem.at[b])

  def gather_wait(q):
    b = q % NB
    pltpu.make_async_copy(h_sh.at[src_v.at[q]], rows_v.at[b],
                          gsem.at[b]).wait()

  def scatter_start(q):
    b = q % NB
    pltpu.async_copy(rows_v.at[b], agg_sh.at[dst_v.at[q]], ssem.at[b],
                     add=True)

  def scatter_wait(q):
    b = q % NB
    pltpu.make_async_copy(rows_v.at[b], agg_sh.at[dst_v.at[q]],
                          ssem.at[b]).wait()

  def block(ib, _):
    pltpu.sync_copy(src_hbm.at[sid, pl.ds(ib * IDXB, IDXB)], src_v)
    pltpu.sync_copy(dst_hbm.at[sid, pl.ds(ib * IDXB, IDXB)], dst_v)
    gather_start(0)
    def step(q, _):
      @pl.when(q + 1 < IDXB)
      def _():
        @pl.when(q + 1 >= NB)
        def _():
          scatter_wait(q + 1 - NB)
        gather_start(q + 1)
      gather_wait(q)
      scatter_start(q)
      if with_deg:
        @pl.when(q % 2 == cid)
        def _():
          @pl.when(q >= 2)
          def _():
            pltpu.make_async_copy(ones_v, deg_sh.at[dst_v.at[q - 2]],
                                  dsem).wait()
          pltpu.async_copy(ones_v, deg_sh.at[dst_v.at[q]], dsem, add=True)
      return 0
    lax.fori_loop(0, IDXB, step, 0)
    for q in (IDXB - 3, IDXB - 2, IDXB - 1):
      scatter_wait(q)
    if with_deg:
      pltpu.make_async_copy(ones_v, deg_sh.at[dst_v.at[IDXB - 2 + cid]],
                            dsem).wait()
    return 0
  lax.fori_loop(0, NCHUNK // IDXB, block, 0)

  plsc.subcore_barrier()

  pltpu.sync_copy(agg_sh.at[pl.ds(row0, RPT)], agg_out.at[cid, pl.ds(row0, RPT)])
  if with_deg:
    pltpu.sync_copy(deg_sh.at[pl.ds(row0, RPT)],
                    deg_out.at[cid, pl.ds(row0, RPT)])


def _make_sc_agg(with_deg):
  mesh = plsc.VectorSubcoreMesh(core_axis_name="c", subcore_axis_name="s")
  out_type = [jax.ShapeDtypeStruct((NC, NP, HH), jnp.float32)]
  scratch = [
      pltpu.VMEM_SHARED((NP, HH), jnp.float32),
      pltpu.VMEM_SHARED((NP, HH), jnp.float32),
  ]
  if with_deg:
    out_type.append(jax.ShapeDtypeStruct((NC, NP, DW), jnp.float32))
    scratch.append(pltpu.VMEM_SHARED((NP, DW), jnp.float32))
  scratch += [
      pltpu.VMEM((IDXB, K), jnp.int32),
      pltpu.VMEM((IDXB, K), jnp.int32),
      pltpu.VMEM((3, K, HH), jnp.float32),
      pltpu.VMEM((ZR, HH), jnp.float32),
  ]
  if with_deg:
    scratch += [
        pltpu.VMEM((ZR, DW), jnp.float32),
        pltpu.VMEM((K, DW), jnp.float32),
    ]
  scratch.append(pltpu.SemaphoreType.DMA((3,)))
  scratch.append(pltpu.SemaphoreType.DMA((3,)))
  if with_deg:
    scratch.append(pltpu.SemaphoreType.DMA)
  return pl.kernel(
      functools.partial(_sc_agg_body, with_deg),
      out_type=tuple(out_type),
      mesh=mesh,
      scratch_types=tuple(scratch),
      compiler_params=pltpu.CompilerParams(use_tc_tiling_on_sc=False),
  )


def _pre_body(x_ref, w_ref, b_ref, hf_ref, hs_ref):
  h = jnp.dot(x_ref[...], w_ref[...], preferred_element_type=jnp.float32)
  h = jnp.maximum(h + b_ref[...], 0.0)
  hf_ref[...] = h
  hs_ref[0] = h[:, :HH]
  hs_ref[1] = h[:, HH:]


def _conv_body(head, *refs):
  if head:
    (agg_ref, deg_ref, h_ref, llw_ref, llb_ref, lrw_ref, g_ref, b_ref,
     hw_ref, hb_ref, hf_ref, y_ref) = refs
  else:
    (agg_ref, deg_ref, h_ref, llw_ref, llb_ref, lrw_ref, g_ref, b_ref,
     hf_ref, hs_ref) = refs
  deg = deg_ref[0][:, 0:1] + deg_ref[1][:, 0:1]
  inv = 1.0 / jnp.maximum(deg, 1.0)
  llw = llw_ref[...]
  z = jnp.dot(agg_ref[0] * inv, llw[:HH, :], preferred_element_type=jnp.float32)
  z = z + jnp.dot(agg_ref[1] * inv, llw[HH:, :], preferred_element_type=jnp.float32)
  z = z + jnp.dot(h_ref[...], lrw_ref[...], preferred_element_type=jnp.float32)
  z = z + llb_ref[...]
  bn_scale = g_ref[...] * (1.0 / jnp.sqrt(1.0 + EPS))
  h = jnp.maximum(z * bn_scale + b_ref[...], 0.0)
  hf_ref[...] = h
  if head:
    y_ref[...] = jnp.dot(h, hw_ref[...], preferred_element_type=jnp.float32) + hb_ref[...]
  else:
    hs_ref[0] = h[:, :HH]
    hs_ref[1] = h[:, HH:]


_row_spec = pl.BlockSpec((ROWBLK, H), lambda i: (i, 0))
_split_spec = pl.BlockSpec((NC, ROWBLK, HH), lambda i: (0, i, 0))
_full_spec = pl.BlockSpec((H, H), lambda i: (0, 0))
_vec_spec = pl.BlockSpec((1, H), lambda i: (0, 0))


def _tc_pre(x, w, b):
  return pl.pallas_call(
      _pre_body,
      grid=(N // ROWBLK,),
      in_specs=[_row_spec, _full_spec, _vec_spec],
      out_specs=[_row_spec, _split_spec],
      out_shape=[
          jax.ShapeDtypeStruct((N, H), jnp.float32),
          jax.ShapeDtypeStruct((NC, NP, HH), jnp.float32),
      ],
  )(x, w, b)


def _tc_conv(agg, deg, h, llw, llb, lrw, g, b):
  return pl.pallas_call(
      functools.partial(_conv_body, False),
      grid=(N // ROWBLK,),
      in_specs=[
          _split_spec,
          pl.BlockSpec((NC, ROWBLK, DW), lambda i: (0, i, 0)),
          _row_spec, _full_spec, _vec_spec, _full_spec, _vec_spec, _vec_spec,
      ],
      out_specs=[_row_spec, _split_spec],
      out_shape=[
          jax.ShapeDtypeStruct((N, H), jnp.float32),
          jax.ShapeDtypeStruct((NC, NP, HH), jnp.float32),
      ],
  )(agg, deg, h, llw, llb, lrw, g, b)


def _tc_conv_head(agg, deg, h, llw, llb, lrw, g, b, hw, hb):
  return pl.pallas_call(
      functools.partial(_conv_body, True),
      grid=(N // ROWBLK,),
      in_specs=[
          _split_spec,
          pl.BlockSpec((NC, ROWBLK, DW), lambda i: (0, i, 0)),
          _row_spec, _full_spec, _vec_spec, _full_spec, _vec_spec, _vec_spec,
          pl.BlockSpec((H, 1), lambda i: (0, 0)),
          pl.BlockSpec((1, 1), lambda i: (0, 0)),
      ],
      out_specs=[_row_spec, pl.BlockSpec((ROWBLK, 1), lambda i: (i, 0))],
      out_shape=[
          jax.ShapeDtypeStruct((N, H), jnp.float32),
          jax.ShapeDtypeStruct((N, 1), jnp.float32),
      ],
  )(agg, deg, h, llw, llb, lrw, g, b, hw, hb)


def kernel(x, edge_index, pre_w, pre_b, c0_ll_w, c0_ll_b, c0_lr_w,
           c1_ll_w, c1_ll_b, c1_lr_w, n0_g, n0_b, n1_g, n1_b,
           head_w, head_b):
  pad = EP - E
  src = jnp.concatenate(
      [edge_index[0], jnp.zeros((pad,), jnp.int32)]).reshape(NS, NCHUNK, K)
  dst = jnp.concatenate(
      [edge_index[1], jnp.full((pad,), PAD_DST, jnp.int32)]).reshape(NS, NCHUNK, K)

  h0, h0s = _tc_pre(x, pre_w, pre_b.reshape(1, H))

  agg0, deg = _make_sc_agg(True)(h0s, src, dst)
  h1, h1s = _tc_conv(agg0, deg, h0, c0_ll_w, c0_ll_b.reshape(1, H),
                     c0_lr_w, n0_g.reshape(1, H), n0_b.reshape(1, H))

  (agg1,) = _make_sc_agg(False)(h1s, src, dst)
  h2, y = _tc_conv_head(agg1, deg, h1, c1_ll_w, c1_ll_b.reshape(1, H),
                        c1_lr_w, n1_g.reshape(1, H), n1_b.reshape(1, H),
                        head_w, head_b.reshape(1, 1))
  return (y[:, 0], h2)


# speedup vs baseline: 8.0803x; 1.0196x over previous
import functools

import jax
import jax.numpy as jnp
from jax import lax
from jax.experimental import pallas as pl
from jax.experimental.pallas import tpu as pltpu
from jax.experimental.pallas import tpu_sc as plsc

N = 10000
E = 320000
D = 128
H = 128
EPS = 1e-5

NC = 2
NS = 16
HH = H // NC
NP = 10240
K = 128
NCHUNK = 160
EP = NS * NCHUNK * K
PAD_DST = 10200
RPT = NP // NS
ZR = 64
DW = 16
IDXB = 16

ROWBLK = 2048


def _zero_fill(ref, rows, width):
  z = jnp.zeros((16,), jnp.float32)
  def body(i, _):
    for j in range(width // 16):
      ref[i, pl.ds(j * 16, 16)] = z
    return 0
  lax.fori_loop(0, rows, body, 0)


def _fill_ones(ref, rows):
  o = jnp.ones((16,), jnp.float32)
  def body(i, _):
    ref[i, :] = o
    return 0
  lax.fori_loop(0, rows, body, 0)


def _sc_agg_body(with_deg, *refs):
  if with_deg:
    (h_hbm, src_hbm, dst_hbm, agg_out, deg_out,
     h_sh, agg_sh, deg_sh, src_v, dst_v, rows_v, zbuf, zdeg, ones_v,
     gsem, ssem, dsem) = refs
  else:
    (h_hbm, src_hbm, dst_hbm, agg_out,
     h_sh, agg_sh, src_v, dst_v, rows_v, zbuf,
     gsem, ssem) = refs

  cid = lax.axis_index("c")
  sid = lax.axis_index("s")
  row0 = sid * RPT

  pltpu.sync_copy(h_hbm.at[cid, pl.ds(row0, RPT)], h_sh.at[pl.ds(row0, RPT)])

  _zero_fill(zbuf, ZR, HH)
  for i in range(RPT // ZR):
    pltpu.sync_copy(zbuf, agg_sh.at[pl.ds(row0 + i * ZR, ZR)])
  if with_deg:
    _zero_fill(zdeg, ZR, DW)
    _fill_ones(ones_v, K)
    for i in range(RPT // ZR):
      pltpu.sync_copy(zdeg, deg_sh.at[pl.ds(row0 + i * ZR, ZR)])

  plsc.subcore_barrier()

  NB = 3

  def gather_start(q):
    b = q % NB
    pltpu.async_copy(h_sh.at[src_v.at[q]], rows_v.at[b], gsem.at[b])

  def gather_wait(q):
    b = q % NB
    pltpu.make_async_copy(h_sh.at[src_v.at[q]], rows_v.at[b],
                          gsem.at[b]).wait()

  def scatter_start(q):
    b = q % NB
    pltpu.async_copy(rows_v.at[b], agg_sh.at[dst_v.at[q]], ssem.at[b],
                     add=True)

  def scatter_wait(q):
    b = q % NB
    pltpu.make_async_copy(rows_v.at[b], agg_sh.at[dst_v.at[q]],
                          ssem.at[b]).wait()

  def block(ib, _):
    pltpu.sync_copy(src_hbm.at[sid, pl.ds(ib * IDXB, IDXB)], src_v)
    pltpu.sync_copy(dst_hbm.at[sid, pl.ds(ib * IDXB, IDXB)], dst_v)
    gather_start(0)
    def step(q, _):
      @pl.when(q + 1 < IDXB)
      def _():
        @pl.when(q + 1 >= NB)
        def _():
          scatter_wait(q + 1 - NB)
        gather_start(q + 1)
      gather_wait(q)
      scatter_start(q)
      if with_deg:
        @pl.when(q % 2 == cid)
        def _():
          @pl.when(q >= 2)
          def _():
            pltpu.make_async_copy(ones_v, deg_sh.at[dst_v.at[q - 2]],
                                  dsem).wait()
          pltpu.async_copy(ones_v, deg_sh.at[dst_v.at[q]], dsem, add=True)
      return 0
    lax.fori_loop(0, IDXB, step, 0)
    for q in (IDXB - 3, IDXB - 2, IDXB - 1):
      scatter_wait(q)
    if with_deg:
      pltpu.make_async_copy(ones_v, deg_sh.at[dst_v.at[IDXB - 2 + cid]],
                            dsem).wait()
    return 0
  lax.fori_loop(0, NCHUNK // IDXB, block, 0)

  plsc.subcore_barrier()

  pltpu.sync_copy(agg_sh.at[pl.ds(row0, RPT)], agg_out.at[cid, pl.ds(row0, RPT)])
  if with_deg:
    pltpu.sync_copy(deg_sh.at[pl.ds(row0, RPT)],
                    deg_out.at[cid, pl.ds(row0, RPT)])


def _make_sc_agg(with_deg):
  mesh = plsc.VectorSubcoreMesh(core_axis_name="c", subcore_axis_name="s")
  out_type = [jax.ShapeDtypeStruct((NC, NP, HH), jnp.float32)]
  scratch = [
      pltpu.VMEM_SHARED((NP, HH), jnp.float32),
      pltpu.VMEM_SHARED((NP, HH), jnp.float32),
  ]
  if with_deg:
    out_type.append(jax.ShapeDtypeStruct((NC, NP, DW), jnp.float32))
    scratch.append(pltpu.VMEM_SHARED((NP, DW), jnp.float32))
  scratch += [
      pltpu.VMEM((IDXB, K), jnp.int32),
      pltpu.VMEM((IDXB, K), jnp.int32),
      pltpu.VMEM((3, K, HH), jnp.float32),
      pltpu.VMEM((ZR, HH), jnp.float32),
  ]
  if with_deg:
    scratch += [
        pltpu.VMEM((ZR, DW), jnp.float32),
        pltpu.VMEM((K, DW), jnp.float32),
    ]
  scratch.append(pltpu.SemaphoreType.DMA((3,)))
  scratch.append(pltpu.SemaphoreType.DMA((3,)))
  if with_deg:
    scratch.append(pltpu.SemaphoreType.DMA)
  return pl.kernel(
      functools.partial(_sc_agg_body, with_deg),
      out_type=tuple(out_type),
      mesh=mesh,
      scratch_types=tuple(scratch),
      compiler_params=pltpu.CompilerParams(use_tc_tiling_on_sc=False),
  )


def _pre_body(x_ref, w_ref, b_ref, hf_ref, hs_ref):
  h = jnp.dot(x_ref[...], w_ref[...], preferred_element_type=jnp.float32)
  h = jnp.maximum(h + b_ref[...], 0.0)
  hf_ref[...] = h
  hs_ref[0] = h[:, :HH]
  hs_ref[1] = h[:, HH:]


def _conv_body(head, *refs):
  if head:
    (agg_ref, deg_ref, h_ref, llw_ref, llb_ref, lrw_ref, g_ref, b_ref,
     hw_ref, hb_ref, hf_ref, y_ref) = refs
  else:
    (agg_ref, deg_ref, h_ref, llw_ref, llb_ref, lrw_ref, g_ref, b_ref,
     hf_ref, hs_ref) = refs
  deg = deg_ref[0][:, 0:1] + deg_ref[1][:, 0:1]
  inv = 1.0 / jnp.maximum(deg, 1.0)
  llw = llw_ref[...]
  z = jnp.dot(agg_ref[0] * inv, llw[:HH, :], preferred_element_type=jnp.float32)
  z = z + jnp.dot(agg_ref[1] * inv, llw[HH:, :], preferred_element_type=jnp.float32)
  z = z + jnp.dot(h_ref[...], lrw_ref[...], preferred_element_type=jnp.float32)
  z = z + llb_ref[...]
  bn_scale = g_ref[...] * (1.0 / jnp.sqrt(1.0 + EPS))
  h = jnp.maximum(z * bn_scale + b_ref[...], 0.0)
  hf_ref[...] = h
  if head:
    y_ref[...] = jnp.dot(h, hw_ref[...], preferred_element_type=jnp.float32) + hb_ref[...]
  else:
    hs_ref[0] = h[:, :HH]
    hs_ref[1] = h[:, HH:]


_row_spec = pl.BlockSpec((ROWBLK, H), lambda i: (i, 0))
_split_spec = pl.BlockSpec((NC, ROWBLK, HH), lambda i: (0, i, 0))
_full_spec = pl.BlockSpec((H, H), lambda i: (0, 0))
_vec_spec = pl.BlockSpec((1, H), lambda i: (0, 0))


def _tc_pre(x, w, b):
  return pl.pallas_call(
      _pre_body,
      grid=(NP // ROWBLK,),
      in_specs=[_row_spec, _full_spec, _vec_spec],
      out_specs=[_row_spec, _split_spec],
      out_shape=[
          jax.ShapeDtypeStruct((N, H), jnp.float32),
          jax.ShapeDtypeStruct((NC, NP, HH), jnp.float32),
      ],
  )(x, w, b)


def _tc_conv(agg, deg, h, llw, llb, lrw, g, b):
  return pl.pallas_call(
      functools.partial(_conv_body, False),
      grid=(NP // ROWBLK,),
      in_specs=[
          _split_spec,
          pl.BlockSpec((NC, ROWBLK, DW), lambda i: (0, i, 0)),
          _row_spec, _full_spec, _vec_spec, _full_spec, _vec_spec, _vec_spec,
      ],
      out_specs=[_row_spec, _split_spec],
      out_shape=[
          jax.ShapeDtypeStruct((N, H), jnp.float32),
          jax.ShapeDtypeStruct((NC, NP, HH), jnp.float32),
      ],
  )(agg, deg, h, llw, llb, lrw, g, b)


def _tc_conv_head(agg, deg, h, llw, llb, lrw, g, b, hw, hb):
  return pl.pallas_call(
      functools.partial(_conv_body, True),
      grid=(NP // ROWBLK,),
      in_specs=[
          _split_spec,
          pl.BlockSpec((NC, ROWBLK, DW), lambda i: (0, i, 0)),
          _row_spec, _full_spec, _vec_spec, _full_spec, _vec_spec, _vec_spec,
          pl.BlockSpec((H, 1), lambda i: (0, 0)),
          pl.BlockSpec((1, 1), lambda i: (0, 0)),
      ],
      out_specs=[_row_spec, pl.BlockSpec((ROWBLK, 1), lambda i: (i, 0))],
      out_shape=[
          jax.ShapeDtypeStruct((N, H), jnp.float32),
          jax.ShapeDtypeStruct((N, 1), jnp.float32),
      ],
  )(agg, deg, h, llw, llb, lrw, g, b, hw, hb)


def kernel(x, edge_index, pre_w, pre_b, c0_ll_w, c0_ll_b, c0_lr_w,
           c1_ll_w, c1_ll_b, c1_lr_w, n0_g, n0_b, n1_g, n1_b,
           head_w, head_b):
  pad = EP - E
  src = jnp.concatenate(
      [edge_index[0], jnp.zeros((pad,), jnp.int32)]).reshape(NS, NCHUNK, K)
  dst = jnp.concatenate(
      [edge_index[1], jnp.full((pad,), PAD_DST, jnp.int32)]).reshape(NS, NCHUNK, K)

  h0, h0s = _tc_pre(x, pre_w, pre_b.reshape(1, H))

  agg0, deg = _make_sc_agg(True)(h0s, src, dst)
  h1, h1s = _tc_conv(agg0, deg, h0, c0_ll_w, c0_ll_b.reshape(1, H),
                     c0_lr_w, n0_g.reshape(1, H), n0_b.reshape(1, H))

  (agg1,) = _make_sc_agg(False)(h1s, src, dst)
  h2, y = _tc_conv_head(agg1, deg, h1, c1_ll_w, c1_ll_b.reshape(1, H),
                        c1_lr_w, n1_g.reshape(1, H), n1_b.reshape(1, H),
                        head_w, head_b.reshape(1, 1))
  return (y[:, 0], h2)


# speedup vs baseline: 8.2870x; 1.0256x over previous
import functools

import jax
import jax.numpy as jnp
from jax import lax
from jax.experimental import pallas as pl
from jax.experimental.pallas import tpu as pltpu
from jax.experimental.pallas import tpu_sc as plsc

N = 10000
E = 320000
D = 128
H = 128
EPS = 1e-5

NC = 2
NS = 16
HH = H // NC
NP = 10240
K = 128
NCHUNK = 160
EP = NS * NCHUNK * K
PAD_DST = 10200
RPT = NP // NS
ZR = 32
DW = 16
IDXB = 16

ROWBLK = 2048


def _zero_fill(ref, rows, width):
  z = jnp.zeros((16,), jnp.float32)
  def body(i, _):
    for j in range(width // 16):
      ref[i, pl.ds(j * 16, 16)] = z
    return 0
  lax.fori_loop(0, rows, body, 0)


def _fill_ones(ref, rows):
  o = jnp.ones((16,), jnp.float32)
  def body(i, _):
    ref[i, :] = o
    return 0
  lax.fori_loop(0, rows, body, 0)


def _sc_agg_body(with_deg, *refs):
  if with_deg:
    (h_hbm, src_hbm, dst_hbm, agg_out, deg_out,
     h_sh, agg_sh, deg_sh, src_v, dst_v, rows_v, zbuf, zdeg, ones_v,
     gsem, ssem, isem, dsem) = refs
    NB = 3
  else:
    (h_hbm, src_hbm, dst_hbm, agg_out,
     h_sh, agg_sh, src_v, dst_v, rows_v, zbuf,
     gsem, ssem, isem) = refs
    NB = 4

  cid = lax.axis_index("c")
  sid = lax.axis_index("s")
  row0 = sid * RPT

  pltpu.sync_copy(h_hbm.at[cid, pl.ds(row0, RPT)], h_sh.at[pl.ds(row0, RPT)])

  _zero_fill(zbuf, ZR, HH)
  for i in range(RPT // ZR):
    pltpu.sync_copy(zbuf, agg_sh.at[pl.ds(row0 + i * ZR, ZR)])
  if with_deg:
    _zero_fill(zdeg, ZR, DW)
    _fill_ones(ones_v, K)
    for i in range(RPT // ZR):
      pltpu.sync_copy(zdeg, deg_sh.at[pl.ds(row0 + i * ZR, ZR)])

  plsc.subcore_barrier()

  def gather_start(pb, q):
    b = q % NB
    pltpu.async_copy(h_sh.at[src_v.at[pb, q]], rows_v.at[b], gsem.at[b])

  def gather_wait(pb, q):
    b = q % NB
    pltpu.make_async_copy(h_sh.at[src_v.at[pb, q]], rows_v.at[b],
                          gsem.at[b]).wait()

  def scatter_start(pb, q):
    b = q % NB
    pltpu.async_copy(rows_v.at[b], agg_sh.at[dst_v.at[pb, q]], ssem.at[b],
                     add=True)

  def scatter_wait(pb, q):
    b = q % NB
    pltpu.make_async_copy(rows_v.at[b], agg_sh.at[dst_v.at[pb, q]],
                          ssem.at[b]).wait()

  def idx_start(ib, pb):
    pltpu.async_copy(src_hbm.at[sid, pl.ds(ib * IDXB, IDXB)], src_v.at[pb],
                     isem.at[pb])
    pltpu.async_copy(dst_hbm.at[sid, pl.ds(ib * IDXB, IDXB)], dst_v.at[pb],
                     isem.at[pb])

  def idx_wait(ib, pb):
    pltpu.make_async_copy(src_hbm.at[sid, pl.ds(ib * IDXB, IDXB)],
                          src_v.at[pb], isem.at[pb]).wait()
    pltpu.make_async_copy(dst_hbm.at[sid, pl.ds(ib * IDXB, IDXB)],
                          dst_v.at[pb], isem.at[pb]).wait()

  NBLK = NCHUNK // IDXB
  idx_start(0, 0)
  def block(ib, _):
    pb = ib % 2
    idx_wait(ib, pb)
    @pl.when(ib + 1 < NBLK)
    def _():
      idx_start(ib + 1, (ib + 1) % 2)
    gather_start(pb, 0)
    def step(q, _):
      @pl.when(q + 1 < IDXB)
      def _():
        @pl.when(q + 1 >= NB)
        def _():
          scatter_wait(pb, q + 1 - NB)
        gather_start(pb, q + 1)
      gather_wait(pb, q)
      scatter_start(pb, q)
      if with_deg:
        @pl.when(q % 2 == cid)
        def _():
          @pl.when(q >= 2)
          def _():
            pltpu.make_async_copy(ones_v, deg_sh.at[dst_v.at[pb, q - 2]],
                                  dsem).wait()
          pltpu.async_copy(ones_v, deg_sh.at[dst_v.at[pb, q]], dsem,
                           add=True)
      return 0
    lax.fori_loop(0, IDXB, step, 0)
    for q in range(IDXB - NB, IDXB):
      scatter_wait(pb, q)
    if with_deg:
      pltpu.make_async_copy(ones_v, deg_sh.at[dst_v.at[pb, IDXB - 2 + cid]],
                            dsem).wait()
    return 0
  lax.fori_loop(0, NBLK, block, 0)

  plsc.subcore_barrier()

  pltpu.sync_copy(agg_sh.at[pl.ds(row0, RPT)], agg_out.at[cid, pl.ds(row0, RPT)])
  if with_deg:
    pltpu.sync_copy(deg_sh.at[pl.ds(row0, RPT)],
                    deg_out.at[cid, pl.ds(row0, RPT)])


def _make_sc_agg(with_deg):
  mesh = plsc.VectorSubcoreMesh(core_axis_name="c", subcore_axis_name="s")
  out_type = [jax.ShapeDtypeStruct((NC, NP, HH), jnp.float32)]
  scratch = [
      pltpu.VMEM_SHARED((NP, HH), jnp.float32),
      pltpu.VMEM_SHARED((NP, HH), jnp.float32),
  ]
  if with_deg:
    out_type.append(jax.ShapeDtypeStruct((NC, NP, DW), jnp.float32))
    scratch.append(pltpu.VMEM_SHARED((NP, DW), jnp.float32))
  nb = 3 if with_deg else 4
  scratch += [
      pltpu.VMEM((2, IDXB, K), jnp.int32),
      pltpu.VMEM((2, IDXB, K), jnp.int32),
      pltpu.VMEM((nb, K, HH), jnp.float32),
      pltpu.VMEM((ZR, HH), jnp.float32),
  ]
  if with_deg:
    scratch += [
        pltpu.VMEM((ZR, DW), jnp.float32),
        pltpu.VMEM((K, DW), jnp.float32),
    ]
  scratch.append(pltpu.SemaphoreType.DMA((nb,)))
  scratch.append(pltpu.SemaphoreType.DMA((nb,)))
  scratch.append(pltpu.SemaphoreType.DMA((2,)))
  if with_deg:
    scratch.append(pltpu.SemaphoreType.DMA)
  return pl.kernel(
      functools.partial(_sc_agg_body, with_deg),
      out_type=tuple(out_type),
      mesh=mesh,
      scratch_types=tuple(scratch),
      compiler_params=pltpu.CompilerParams(use_tc_tiling_on_sc=False),
  )


def _pre_body(x_ref, w_ref, b_ref, hf_ref, hs_ref):
  h = jnp.dot(x_ref[...], w_ref[...], preferred_element_type=jnp.float32)
  h = jnp.maximum(h + b_ref[...], 0.0)
  hf_ref[...] = h
  hs_ref[0] = h[:, :HH]
  hs_ref[1] = h[:, HH:]


def _conv_body(head, *refs):
  if head:
    (agg_ref, deg_ref, h_ref, llw_ref, llb_ref, lrw_ref, g_ref, b_ref,
     hw_ref, hb_ref, hf_ref, y_ref) = refs
  else:
    (agg_ref, deg_ref, h_ref, llw_ref, llb_ref, lrw_ref, g_ref, b_ref,
     hf_ref, hs_ref) = refs
  deg = deg_ref[0][:, 0:1] + deg_ref[1][:, 0:1]
  inv = 1.0 / jnp.maximum(deg, 1.0)
  llw = llw_ref[...]
  z = jnp.dot(agg_ref[0] * inv, llw[:HH, :], preferred_element_type=jnp.float32)
  z = z + jnp.dot(agg_ref[1] * inv, llw[HH:, :], preferred_element_type=jnp.float32)
  z = z + jnp.dot(h_ref[...], lrw_ref[...], preferred_element_type=jnp.float32)
  z = z + llb_ref[...]
  bn_scale = g_ref[...] * (1.0 / jnp.sqrt(1.0 + EPS))
  h = jnp.maximum(z * bn_scale + b_ref[...], 0.0)
  hf_ref[...] = h
  if head:
    y_ref[...] = jnp.dot(h, hw_ref[...], preferred_element_type=jnp.float32) + hb_ref[...]
  else:
    hs_ref[0] = h[:, :HH]
    hs_ref[1] = h[:, HH:]


_row_spec = pl.BlockSpec((ROWBLK, H), lambda i: (i, 0))
_split_spec = pl.BlockSpec((NC, ROWBLK, HH), lambda i: (0, i, 0))
_full_spec = pl.BlockSpec((H, H), lambda i: (0, 0))
_vec_spec = pl.BlockSpec((1, H), lambda i: (0, 0))


def _tc_pre(x, w, b):
  return pl.pallas_call(
      _pre_body,
      grid=(NP // ROWBLK,),
      in_specs=[_row_spec, _full_spec, _vec_spec],
      out_specs=[_row_spec, _split_spec],
      out_shape=[
          jax.ShapeDtypeStruct((N, H), jnp.float32),
          jax.ShapeDtypeStruct((NC, NP, HH), jnp.float32),
      ],
  )(x, w, b)


def _tc_conv(agg, deg, h, llw, llb, lrw, g, b):
  return pl.pallas_call(
      functools.partial(_conv_body, False),
      grid=(NP // ROWBLK,),
      in_specs=[
          _split_spec,
          pl.BlockSpec((NC, ROWBLK, DW), lambda i: (0, i, 0)),
          _row_spec, _full_spec, _vec_spec, _full_spec, _vec_spec, _vec_spec,
      ],
      out_specs=[_row_spec, _split_spec],
      out_shape=[
          jax.ShapeDtypeStruct((N, H), jnp.float32),
          jax.ShapeDtypeStruct((NC, NP, HH), jnp.float32),
      ],
  )(agg, deg, h, llw, llb, lrw, g, b)


def _tc_conv_head(agg, deg, h, llw, llb, lrw, g, b, hw, hb):
  return pl.pallas_call(
      functools.partial(_conv_body, True),
      grid=(NP // ROWBLK,),
      in_specs=[
          _split_spec,
          pl.BlockSpec((NC, ROWBLK, DW), lambda i: (0, i, 0)),
          _row_spec, _full_spec, _vec_spec, _full_spec, _vec_spec, _vec_spec,
          pl.BlockSpec((H, 1), lambda i: (0, 0)),
          pl.BlockSpec((1, 1), lambda i: (0, 0)),
      ],
      out_specs=[_row_spec, pl.BlockSpec((ROWBLK, 1), lambda i: (i, 0))],
      out_shape=[
          jax.ShapeDtypeStruct((N, H), jnp.float32),
          jax.ShapeDtypeStruct((N, 1), jnp.float32),
      ],
  )(agg, deg, h, llw, llb, lrw, g, b, hw, hb)


def kernel(x, edge_index, pre_w, pre_b, c0_ll_w, c0_ll_b, c0_lr_w,
           c1_ll_w, c1_ll_b, c1_lr_w, n0_g, n0_b, n1_g, n1_b,
           head_w, head_b):
  pad = EP - E
  src = jnp.concatenate(
      [edge_index[0], jnp.zeros((pad,), jnp.int32)]).reshape(NS, NCHUNK, K)
  dst = jnp.concatenate(
      [edge_index[1], jnp.full((pad,), PAD_DST, jnp.int32)]).reshape(NS, NCHUNK, K)

  h0, h0s = _tc_pre(x, pre_w, pre_b.reshape(1, H))

  agg0, deg = _make_sc_agg(True)(h0s, src, dst)
  h1, h1s = _tc_conv(agg0, deg, h0, c0_ll_w, c0_ll_b.reshape(1, H),
                     c0_lr_w, n0_g.reshape(1, H), n0_b.reshape(1, H))

  (agg1,) = _make_sc_agg(False)(h1s, src, dst)
  h2, y = _tc_conv_head(agg1, deg, h1, c1_ll_w, c1_ll_b.reshape(1, H),
                        c1_lr_w, n1_g.reshape(1, H), n1_b.reshape(1, H),
                        head_w, head_b.reshape(1, 1))
  return (y[:, 0], h2)


# speedup vs baseline: 9.2332x; 1.1142x over previous
import functools

import jax
import jax.numpy as jnp
from jax import lax
from jax.experimental import pallas as pl
from jax.experimental.pallas import tpu as pltpu
from jax.experimental.pallas import tpu_sc as plsc

N = 10000
E = 320000
D = 128
H = 128
EPS = 1e-5

NC = 2
NS = 16
HH = H // NC
NP = 10240
K = 128
NCH = E // K
CPT = 156
IDXB = 12
NBLK = CPT // IDXB
RPT = NP // NS
ZR = 32
DW = 16

ROWBLK = 2048


def _zero_fill(ref, rows, width):
  z = jnp.zeros((16,), jnp.float32)
  def body(i, _):
    for j in range(width // 16):
      ref[i, pl.ds(j * 16, 16)] = z
    return 0
  lax.fori_loop(0, rows, body, 0)


def _fill_ones(ref, rows):
  o = jnp.ones((16,), jnp.float32)
  def body(i, _):
    ref[i, :] = o
    return 0
  lax.fori_loop(0, rows, body, 0)


def _sc_agg_body(with_deg, *refs):
  if with_deg:
    (h_hbm, e_hbm, agg_out, deg_out,
     h_sh, agg_sh, deg_sh, src_v, dst_v, rows_v, zbuf, zdeg, ones_v,
     gsem, ssem, isem, dsem) = refs
    NB = 3
  else:
    (h_hbm, e_hbm, agg_out,
     h_sh, agg_sh, src_v, dst_v, rows_v, zbuf,
     gsem, ssem, isem) = refs
    NB = 4

  cid = lax.axis_index("c")
  sid = lax.axis_index("s")
  row0 = sid * RPT
  col0 = cid * HH
  base = sid * CPT

  pltpu.sync_copy(h_hbm.at[pl.ds(row0, RPT), pl.ds(col0, HH)],
                  h_sh.at[pl.ds(row0, RPT)])

  _zero_fill(zbuf, ZR, HH)
  for i in range(RPT // ZR):
    pltpu.sync_copy(zbuf, agg_sh.at[pl.ds(row0 + i * ZR, ZR)])
  if with_deg:
    _zero_fill(zdeg, ZR, DW)
    _fill_ones(ones_v, K)
    for i in range(RPT // ZR):
      pltpu.sync_copy(zdeg, deg_sh.at[pl.ds(row0 + i * ZR, ZR)])

  plsc.subcore_barrier()

  def gather_start(pb, q):
    b = q % NB
    pltpu.async_copy(h_sh.at[src_v.at[pb, q]], rows_v.at[b], gsem.at[b])

  def gather_wait(pb, q):
    b = q % NB
    pltpu.make_async_copy(h_sh.at[src_v.at[pb, q]], rows_v.at[b],
                          gsem.at[b]).wait()

  def scatter_start(pb, q):
    b = q % NB
    pltpu.async_copy(rows_v.at[b], agg_sh.at[dst_v.at[pb, q]], ssem.at[b],
                     add=True)

  def scatter_wait(pb, q):
    b = q % NB
    pltpu.make_async_copy(rows_v.at[b], agg_sh.at[dst_v.at[pb, q]],
                          ssem.at[b]).wait()

  def idx_start(ib, pb):
    pltpu.async_copy(e_hbm.at[0, pl.ds(base + ib * IDXB, IDXB)],
                     src_v.at[pb], isem.at[pb])
    pltpu.async_copy(e_hbm.at[1, pl.ds(base + ib * IDXB, IDXB)],
                     dst_v.at[pb], isem.at[pb])

  def idx_wait(ib, pb):
    pltpu.make_async_copy(e_hbm.at[0, pl.ds(base + ib * IDXB, IDXB)],
                          src_v.at[pb], isem.at[pb]).wait()
    pltpu.make_async_copy(e_hbm.at[1, pl.ds(base + ib * IDXB, IDXB)],
                          dst_v.at[pb], isem.at[pb]).wait()

  idx_start(0, 0)
  def block(ib, _):
    pb = ib % 2
    idx_wait(ib, pb)
    @pl.when(ib + 1 < NBLK)
    def _():
      idx_start(ib + 1, (ib + 1) % 2)
    gather_start(pb, 0)
    def step(q, _):
      @pl.when(q + 1 < IDXB)
      def _():
        @pl.when(q + 1 >= NB)
        def _():
          scatter_wait(pb, q + 1 - NB)
        gather_start(pb, q + 1)
      gather_wait(pb, q)
      scatter_start(pb, q)
      if with_deg:
        @pl.when(q % 2 == cid)
        def _():
          @pl.when(q >= 2)
          def _():
            pltpu.make_async_copy(ones_v, deg_sh.at[dst_v.at[pb, q - 2]],
                                  dsem).wait()
          pltpu.async_copy(ones_v, deg_sh.at[dst_v.at[pb, q]], dsem,
                           add=True)
      return 0
    lax.fori_loop(0, IDXB, step, 0)
    for q in range(IDXB - NB, IDXB):
      scatter_wait(pb, q)
    if with_deg:
      pltpu.make_async_copy(ones_v, deg_sh.at[dst_v.at[pb, IDXB - 2 + cid]],
                            dsem).wait()
    return 0
  lax.fori_loop(0, NBLK, block, 0)

  @pl.when(sid < NCH - NS * CPT)
  def _():
    xc = NS * CPT + sid
    pltpu.sync_copy(e_hbm.at[0, pl.ds(xc, 1)], src_v.at[0, pl.ds(0, 1)])
    pltpu.sync_copy(e_hbm.at[1, pl.ds(xc, 1)], dst_v.at[0, pl.ds(0, 1)])
    pltpu.async_copy(h_sh.at[src_v.at[0, 0]], rows_v.at[0],
                     gsem.at[0]).wait()
    pltpu.sync_copy(rows_v.at[0], agg_sh.at[dst_v.at[0, 0]], add=True)
    if with_deg:
      @pl.when(cid == 0)
      def _():
        pltpu.sync_copy(ones_v, deg_sh.at[dst_v.at[0, 0]], add=True)

  plsc.subcore_barrier()

  pltpu.sync_copy(agg_sh.at[pl.ds(row0, RPT)],
                  agg_out.at[pl.ds(row0, RPT), pl.ds(col0, HH)])
  if with_deg:
    pltpu.sync_copy(deg_sh.at[pl.ds(row0, RPT)],
                    deg_out.at[cid, pl.ds(row0, RPT)])


def _make_sc_agg(with_deg):
  mesh = plsc.VectorSubcoreMesh(core_axis_name="c", subcore_axis_name="s")
  out_type = [jax.ShapeDtypeStruct((NP, H), jnp.float32)]
  scratch = [
      pltpu.VMEM_SHARED((NP, HH), jnp.float32),
      pltpu.VMEM_SHARED((NP, HH), jnp.float32),
  ]
  if with_deg:
    out_type.append(jax.ShapeDtypeStruct((NC, NP, DW), jnp.float32))
    scratch.append(pltpu.VMEM_SHARED((NP, DW), jnp.float32))
  nb = 3 if with_deg else 4
  scratch += [
      pltpu.VMEM((2, IDXB, K), jnp.int32),
      pltpu.VMEM((2, IDXB, K), jnp.int32),
      pltpu.VMEM((nb, K, HH), jnp.float32),
      pltpu.VMEM((ZR, HH), jnp.float32),
  ]
  if with_deg:
    scratch += [
        pltpu.VMEM((ZR, DW), jnp.float32),
        pltpu.VMEM((K, DW), jnp.float32),
    ]
  scratch.append(pltpu.SemaphoreType.DMA((nb,)))
  scratch.append(pltpu.SemaphoreType.DMA((nb,)))
  scratch.append(pltpu.SemaphoreType.DMA((2,)))
  if with_deg:
    scratch.append(pltpu.SemaphoreType.DMA)
  return pl.kernel(
      functools.partial(_sc_agg_body, with_deg),
      out_type=tuple(out_type),
      mesh=mesh,
      scratch_types=tuple(scratch),
      compiler_params=pltpu.CompilerParams(use_tc_tiling_on_sc=False),
  )


def _pre_body(x_ref, w_ref, b_ref, hf_ref):
  h = jnp.dot(x_ref[...], w_ref[...], preferred_element_type=jnp.float32)
  hf_ref[...] = jnp.maximum(h + b_ref[...], 0.0)


def _conv_body(head, *refs):
  if head:
    (agg_ref, deg_ref, h_ref, llw_ref, llb_ref, lrw_ref, g_ref, b_ref,
     hw_ref, hb_ref, hf_ref, y_ref) = refs
  else:
    (agg_ref, deg_ref, h_ref, llw_ref, llb_ref, lrw_ref, g_ref, b_ref,
     hf_ref) = refs
  deg = deg_ref[0][:, 0:1] + deg_ref[1][:, 0:1]
  inv = 1.0 / jnp.maximum(deg, 1.0)
  z = jnp.dot(agg_ref[...] * inv, llw_ref[...],
              preferred_element_type=jnp.float32)
  z = z + jnp.dot(h_ref[...], lrw_ref[...], preferred_element_type=jnp.float32)
  z = z + llb_ref[...]
  bn_scale = g_ref[...] * (1.0 / jnp.sqrt(1.0 + EPS))
  h = jnp.maximum(z * bn_scale + b_ref[...], 0.0)
  hf_ref[...] = h
  if head:
    y_ref[...] = jnp.dot(h, hw_ref[...], preferred_element_type=jnp.float32) + hb_ref[...]


_row_spec = pl.BlockSpec((ROWBLK, H), lambda i: (i, 0))
_deg_spec = pl.BlockSpec((NC, ROWBLK, DW), lambda i: (0, i, 0))
_full_spec = pl.BlockSpec((H, H), lambda i: (0, 0))
_vec_spec = pl.BlockSpec((1, H), lambda i: (0, 0))


def _tc_pre(x, w, b):
  return pl.pallas_call(
      _pre_body,
      grid=(NP // ROWBLK,),
      in_specs=[_row_spec, _full_spec, _vec_spec],
      out_specs=_row_spec,
      out_shape=jax.ShapeDtypeStruct((NP, H), jnp.float32),
  )(x, w, b)


def _tc_conv(agg, deg, h, llw, llb, lrw, g, b):
  return pl.pallas_call(
      functools.partial(_conv_body, False),
      grid=(NP // ROWBLK,),
      in_specs=[
          _row_spec, _deg_spec,
          _row_spec, _full_spec, _vec_spec, _full_spec, _vec_spec, _vec_spec,
      ],
      out_specs=_row_spec,
      out_shape=jax.ShapeDtypeStruct((NP, H), jnp.float32),
  )(agg, deg, h, llw, llb, lrw, g, b)


def _tc_conv_head(agg, deg, h, llw, llb, lrw, g, b, hw, hb):
  return pl.pallas_call(
      functools.partial(_conv_body, True),
      grid=(NP // ROWBLK,),
      in_specs=[
          _row_spec, _deg_spec,
          _row_spec, _full_spec, _vec_spec, _full_spec, _vec_spec, _vec_spec,
          pl.BlockSpec((H, 1), lambda i: (0, 0)),
          pl.BlockSpec((1, 1), lambda i: (0, 0)),
      ],
      out_specs=[_row_spec, pl.BlockSpec((ROWBLK, 1), lambda i: (i, 0))],
      out_shape=[
          jax.ShapeDtypeStruct((N, H), jnp.float32),
          jax.ShapeDtypeStruct((N, 1), jnp.float32),
      ],
  )(agg, deg, h, llw, llb, lrw, g, b, hw, hb)


def kernel(x, edge_index, pre_w, pre_b, c0_ll_w, c0_ll_b, c0_lr_w,
           c1_ll_w, c1_ll_b, c1_lr_w, n0_g, n0_b, n1_g, n1_b,
           head_w, head_b):
  e3 = edge_index.reshape(2, NCH, K)

  h0 = _tc_pre(x, pre_w, pre_b.reshape(1, H))

  agg0, deg = _make_sc_agg(True)(h0, e3)
  h1 = _tc_conv(agg0, deg, h0, c0_ll_w, c0_ll_b.reshape(1, H),
                c0_lr_w, n0_g.reshape(1, H), n0_b.reshape(1, H))

  (agg1,) = _make_sc_agg(False)(h1, e3)
  h2, y = _tc_conv_head(agg1, deg, h1, c1_ll_w, c1_ll_b.reshape(1, H),
                        c1_lr_w, n1_g.reshape(1, H), n1_b.reshape(1, H),
                        head_w, head_b.reshape(1, 1))
  return (y[:, 0], h2)


# speedup vs baseline: 9.3549x; 1.0132x over previous
import functools

import jax
import jax.numpy as jnp
from jax import lax
from jax.experimental import pallas as pl
from jax.experimental.pallas import tpu as pltpu
from jax.experimental.pallas import tpu_sc as plsc

N = 10000
E = 320000
D = 128
H = 128
EPS = 1e-5

NC = 2
NS = 16
HH = H // NC
NP = 10240
K = 128
NCH = E // K
CPT = 156
IDXB = 12
NBLK = CPT // IDXB
RPT = NP // NS
ZR = 32
ZR4 = 128
DW = 16

ROWBLK = 2048


def _zero_fill(ref, rows, width):
  z = jnp.zeros((16,), jnp.float32)
  def body(i, _):
    for j in range(width // 16):
      ref[i, pl.ds(j * 16, 16)] = z
    return 0
  lax.fori_loop(0, rows, body, 0)


def _fill_ones(ref, rows):
  o = jnp.ones((16,), jnp.float32)
  def body(i, _):
    ref[i, :] = o
    return 0
  lax.fori_loop(0, rows, body, 0)


def _sc_agg_body(with_deg, *refs):
  if with_deg:
    (h_hbm, e_hbm, agg_out, deg_out,
     h_sh, agg_sh, deg_sh, src_v, dst_v, rows_v, zbuf, zdeg, ones_v,
     gsem, ssem, isem, dsem) = refs
    NB = 3
  else:
    (h_hbm, e_hbm, agg_out,
     h_sh, agg_sh, src_v, dst_v, rows_v, zbuf,
     gsem, ssem, isem) = refs
    NB = 4

  cid = lax.axis_index("c")
  sid = lax.axis_index("s")
  row0 = sid * RPT
  col0 = cid * HH
  base = sid * CPT

  pltpu.sync_copy(h_hbm.at[pl.ds(row0, RPT), pl.ds(col0, HH)],
                  h_sh.at[pl.ds(row0, RPT)])

  _zero_fill(zbuf, ZR, HH)
  for i in range(RPT // ZR):
    pltpu.sync_copy(zbuf, agg_sh.at[pl.ds(row0 + i * ZR, ZR)])
  if with_deg:
    _zero_fill(zdeg, ZR4, DW)
    _fill_ones(ones_v, K)
    for i in range(RPT // ZR4):
      pltpu.sync_copy(zdeg, deg_sh.at[pl.ds(row0 + i * ZR4, ZR4)])

  plsc.subcore_barrier()

  def gather_start(pb, q):
    b = q % NB
    pltpu.async_copy(h_sh.at[src_v.at[pb, q]], rows_v.at[b], gsem.at[b])

  def gather_wait(pb, q):
    b = q % NB
    pltpu.make_async_copy(h_sh.at[src_v.at[pb, q]], rows_v.at[b],
                          gsem.at[b]).wait()

  def scatter_start(pb, q):
    b = q % NB
    pltpu.async_copy(rows_v.at[b], agg_sh.at[dst_v.at[pb, q]], ssem.at[b],
                     add=True)

  def scatter_wait(pb, q):
    b = q % NB
    pltpu.make_async_copy(rows_v.at[b], agg_sh.at[dst_v.at[pb, q]],
                          ssem.at[b]).wait()

  def idx_start(ib, pb):
    pltpu.async_copy(e_hbm.at[0, pl.ds(base + ib * IDXB, IDXB)],
                     src_v.at[pb], isem.at[pb])
    pltpu.async_copy(e_hbm.at[1, pl.ds(base + ib * IDXB, IDXB)],
                     dst_v.at[pb], isem.at[pb])

  def idx_wait(ib, pb):
    pltpu.make_async_copy(e_hbm.at[0, pl.ds(base + ib * IDXB, IDXB)],
                          src_v.at[pb], isem.at[pb]).wait()
    pltpu.make_async_copy(e_hbm.at[1, pl.ds(base + ib * IDXB, IDXB)],
                          dst_v.at[pb], isem.at[pb]).wait()

  idx_start(0, 0)
  def block(ib, _):
    pb = ib % 2
    idx_wait(ib, pb)
    @pl.when(ib + 1 < NBLK)
    def _():
      idx_start(ib + 1, (ib + 1) % 2)
    gather_start(pb, 0)
    def step(q, _):
      @pl.when(q + 1 < IDXB)
      def _():
        @pl.when(q + 1 >= NB)
        def _():
          scatter_wait(pb, q + 1 - NB)
        gather_start(pb, q + 1)
      gather_wait(pb, q)
      scatter_start(pb, q)
      if with_deg:
        @pl.when(q % 2 == cid)
        def _():
          d = (q // 2) % 2
          @pl.when(q >= 4)
          def _():
            pltpu.make_async_copy(ones_v, deg_sh.at[dst_v.at[pb, q - 4]],
                                  dsem.at[d]).wait()
          pltpu.async_copy(ones_v, deg_sh.at[dst_v.at[pb, q]], dsem.at[d],
                           add=True)
      return 0
    lax.fori_loop(0, IDXB, step, 0)
    for q in range(IDXB - NB, IDXB):
      scatter_wait(pb, q)
    if with_deg:
      for qq in (IDXB - 4, IDXB - 2):
        q = qq + cid
        pltpu.make_async_copy(ones_v, deg_sh.at[dst_v.at[pb, q]],
                              dsem.at[(q // 2) % 2]).wait()
    return 0
  lax.fori_loop(0, NBLK, block, 0)

  @pl.when(sid < NCH - NS * CPT)
  def _():
    xc = NS * CPT + sid
    pltpu.sync_copy(e_hbm.at[0, pl.ds(xc, 1)], src_v.at[0, pl.ds(0, 1)])
    pltpu.sync_copy(e_hbm.at[1, pl.ds(xc, 1)], dst_v.at[0, pl.ds(0, 1)])
    pltpu.async_copy(h_sh.at[src_v.at[0, 0]], rows_v.at[0],
                     gsem.at[0]).wait()
    pltpu.sync_copy(rows_v.at[0], agg_sh.at[dst_v.at[0, 0]], add=True)
    if with_deg:
      @pl.when(cid == 0)
      def _():
        pltpu.sync_copy(ones_v, deg_sh.at[dst_v.at[0, 0]], add=True)

  plsc.subcore_barrier()

  pltpu.sync_copy(agg_sh.at[pl.ds(row0, RPT)],
                  agg_out.at[pl.ds(row0, RPT), pl.ds(col0, HH)])
  if with_deg:
    pltpu.sync_copy(deg_sh.at[pl.ds(row0, RPT)],
                    deg_out.at[cid, pl.ds(row0, RPT)])


def _make_sc_agg(with_deg):
  mesh = plsc.VectorSubcoreMesh(core_axis_name="c", subcore_axis_name="s")
  out_type = [jax.ShapeDtypeStruct((NP, H), jnp.float32)]
  scratch = [
      pltpu.VMEM_SHARED((NP, HH), jnp.float32),
      pltpu.VMEM_SHARED((NP, HH), jnp.float32),
  ]
  if with_deg:
    out_type.append(jax.ShapeDtypeStruct((NC, NP, DW), jnp.float32))
    scratch.append(pltpu.VMEM_SHARED((NP, DW), jnp.float32))
  nb = 3 if with_deg else 4
  scratch += [
      pltpu.VMEM((2, IDXB, K), jnp.int32),
      pltpu.VMEM((2, IDXB, K), jnp.int32),
      pltpu.VMEM((nb, K, HH), jnp.float32),
      pltpu.VMEM((ZR, HH), jnp.float32),
  ]
  if with_deg:
    scratch += [
        pltpu.VMEM((ZR4, DW), jnp.float32),
        pltpu.VMEM((K, DW), jnp.float32),
    ]
  scratch.append(pltpu.SemaphoreType.DMA((nb,)))
  scratch.append(pltpu.SemaphoreType.DMA((nb,)))
  scratch.append(pltpu.SemaphoreType.DMA((2,)))
  if with_deg:
    scratch.append(pltpu.SemaphoreType.DMA((2,)))
  return pl.kernel(
      functools.partial(_sc_agg_body, with_deg),
      out_type=tuple(out_type),
      mesh=mesh,
      scratch_types=tuple(scratch),
      compiler_params=pltpu.CompilerParams(use_tc_tiling_on_sc=False),
  )


def _pre_body(x_ref, w_ref, b_ref, hf_ref):
  h = jnp.dot(x_ref[...], w_ref[...], preferred_element_type=jnp.float32)
  hf_ref[...] = jnp.maximum(h + b_ref[...], 0.0)


def _conv_body(head, *refs):
  if head:
    (agg_ref, deg_ref, h_ref, llw_ref, llb_ref, lrw_ref, g_ref, b_ref,
     hw_ref, hb_ref, hf_ref, y_ref) = refs
  else:
    (agg_ref, deg_ref, h_ref, llw_ref, llb_ref, lrw_ref, g_ref, b_ref,
     hf_ref) = refs
  deg = deg_ref[0][:, 0:1] + deg_ref[1][:, 0:1]
  inv = 1.0 / jnp.maximum(deg, 1.0)
  z = jnp.dot(agg_ref[...] * inv, llw_ref[...],
              preferred_element_type=jnp.float32)
  z = z + jnp.dot(h_ref[...], lrw_ref[...], preferred_element_type=jnp.float32)
  z = z + llb_ref[...]
  bn_scale = g_ref[...] * (1.0 / jnp.sqrt(1.0 + EPS))
  h = jnp.maximum(z * bn_scale + b_ref[...], 0.0)
  hf_ref[...] = h
  if head:
    y_ref[...] = lax.dot_general(
        hw_ref[...], h, (((1,), (1,)), ((), ())),
        preferred_element_type=jnp.float32) + hb_ref[...]


_row_spec = pl.BlockSpec((ROWBLK, H), lambda i: (i, 0))
_deg_spec = pl.BlockSpec((NC, ROWBLK, DW), lambda i: (0, i, 0))
_full_spec = pl.BlockSpec((H, H), lambda i: (0, 0))
_vec_spec = pl.BlockSpec((1, H), lambda i: (0, 0))


def _tc_pre(x, w, b):
  return pl.pallas_call(
      _pre_body,
      grid=(NP // ROWBLK,),
      in_specs=[_row_spec, _full_spec, _vec_spec],
      out_specs=_row_spec,
      out_shape=jax.ShapeDtypeStruct((NP, H), jnp.float32),
  )(x, w, b)


def _tc_conv(agg, deg, h, llw, llb, lrw, g, b):
  return pl.pallas_call(
      functools.partial(_conv_body, False),
      grid=(NP // ROWBLK,),
      in_specs=[
          _row_spec, _deg_spec,
          _row_spec, _full_spec, _vec_spec, _full_spec, _vec_spec, _vec_spec,
      ],
      out_specs=_row_spec,
      out_shape=jax.ShapeDtypeStruct((NP, H), jnp.float32),
  )(agg, deg, h, llw, llb, lrw, g, b)


def _tc_conv_head(agg, deg, h, llw, llb, lrw, g, b, hw, hb):
  return pl.pallas_call(
      functools.partial(_conv_body, True),
      grid=(NP // ROWBLK,),
      in_specs=[
          _row_spec, _deg_spec,
          _row_spec, _full_spec, _vec_spec, _full_spec, _vec_spec, _vec_spec,
          pl.BlockSpec((1, H), lambda i: (0, 0)),
          pl.BlockSpec((1, 1), lambda i: (0, 0)),
      ],
      out_specs=[_row_spec, pl.BlockSpec((1, ROWBLK), lambda i: (0, i))],
      out_shape=[
          jax.ShapeDtypeStruct((N, H), jnp.float32),
          jax.ShapeDtypeStruct((1, N), jnp.float32),
      ],
  )(agg, deg, h, llw, llb, lrw, g, b, hw, hb)


def kernel(x, edge_index, pre_w, pre_b, c0_ll_w, c0_ll_b, c0_lr_w,
           c1_ll_w, c1_ll_b, c1_lr_w, n0_g, n0_b, n1_g, n1_b,
           head_w, head_b):
  e3 = edge_index.reshape(2, NCH, K)

  h0 = _tc_pre(x, pre_w, pre_b.reshape(1, H))

  agg0, deg = _make_sc_agg(True)(h0, e3)
  h1 = _tc_conv(agg0, deg, h0, c0_ll_w, c0_ll_b.reshape(1, H),
                c0_lr_w, n0_g.reshape(1, H), n0_b.reshape(1, H))

  (agg1,) = _make_sc_agg(False)(h1, e3)
  h2, y = _tc_conv_head(agg1, deg, h1, c1_ll_w, c1_ll_b.reshape(1, H),
                        c1_lr_w, n1_g.reshape(1, H), n1_b.reshape(1, H),
                        head_w.reshape(1, H), head_b.reshape(1, 1))
  return (y[0], h2)


# speedup vs baseline: 10.9209x; 1.1674x over previous
import functools

import jax
import jax.numpy as jnp
from jax import lax
from jax.experimental import pallas as pl
from jax.experimental.pallas import tpu as pltpu
from jax.experimental.pallas import tpu_sc as plsc

N = 10000
E = 320000
D = 128
H = 128
EPS = 1e-5

NC = 2
NS = 16
HH = H // NC
NP = 10240
K = 128
NCH = E // K
CPT = 156
IDXB = 12
NBLK = CPT // IDXB
RPT = NP // NS
ZR = 32
ZR4 = 128
DW = 16

ROWBLK = 2048


def _zero_fill(ref, rows, width):
  z = jnp.zeros((16,), jnp.float32)
  def body(i, _):
    for j in range(width // 16):
      ref[i, pl.ds(j * 16, 16)] = z
    return 0
  lax.fori_loop(0, rows, body, 0)


def _fill_ones(ref, rows):
  o = jnp.ones((16,), jnp.float32)
  def body(i, _):
    ref[i, :] = o
    return 0
  lax.fori_loop(0, rows, body, 0)


def _sc_agg_body(with_deg, *refs):
  if with_deg:
    (h_hbm, e_hbm, agg_out, deg_out,
     h_sh, agg_sh, deg_sh, src_v, dst_v, rows_v, zbuf, zdeg, ones_v,
     gsem, ssem, isem, dsem) = refs
    NB = 3
  else:
    (h_hbm, e_hbm, agg_out,
     h_sh, agg_sh, src_v, dst_v, rows_v, zbuf,
     gsem, ssem, isem) = refs
    NB = 4

  cid = lax.axis_index("c")
  sid = lax.axis_index("s")
  row0 = sid * RPT
  col0 = cid * HH
  base = sid * CPT

  pltpu.sync_copy(h_hbm.at[pl.ds(row0, RPT), pl.ds(col0, HH)],
                  h_sh.at[pl.ds(row0, RPT)])

  _zero_fill(zbuf, ZR, HH)
  for i in range(RPT // ZR):
    pltpu.sync_copy(zbuf, agg_sh.at[pl.ds(row0 + i * ZR, ZR)])
  if with_deg:
    _zero_fill(zdeg, ZR4, DW)
    _fill_ones(ones_v, K)
    for i in range(RPT // ZR4):
      pltpu.sync_copy(zdeg, deg_sh.at[pl.ds(row0 + i * ZR4, ZR4)])

  plsc.subcore_barrier()

  def gather_start(pb, q):
    b = q % NB
    pltpu.async_copy(h_sh.at[src_v.at[pb, q]], rows_v.at[b], gsem.at[b])

  def gather_wait(pb, q):
    b = q % NB
    pltpu.make_async_copy(h_sh.at[src_v.at[pb, q]], rows_v.at[b],
                          gsem.at[b]).wait()

  def scatter_start(pb, q):
    b = q % NB
    pltpu.async_copy(rows_v.at[b], agg_sh.at[dst_v.at[pb, q]], ssem.at[b],
                     add=True)

  def scatter_wait(pb, q):
    b = q % NB
    pltpu.make_async_copy(rows_v.at[b], agg_sh.at[dst_v.at[pb, q]],
                          ssem.at[b]).wait()

  def idx_start(ib, pb):
    pltpu.async_copy(e_hbm.at[0, pl.ds(base + ib * IDXB, IDXB)],
                     src_v.at[pb], isem.at[pb])
    pltpu.async_copy(e_hbm.at[1, pl.ds(base + ib * IDXB, IDXB)],
                     dst_v.at[pb], isem.at[pb])

  def idx_wait(ib, pb):
    pltpu.make_async_copy(e_hbm.at[0, pl.ds(base + ib * IDXB, IDXB)],
                          src_v.at[pb], isem.at[pb]).wait()
    pltpu.make_async_copy(e_hbm.at[1, pl.ds(base + ib * IDXB, IDXB)],
                          dst_v.at[pb], isem.at[pb]).wait()

  def g_gather_start(g):
    gather_start((g // IDXB) % 2, g % IDXB)

  def g_gather_wait(g):
    gather_wait((g // IDXB) % 2, g % IDXB)

  def g_scatter_start(g):
    scatter_start((g // IDXB) % 2, g % IDXB)

  def g_scatter_wait(g):
    scatter_wait((g // IDXB) % 2, g % IDXB)

  def g_deg_ref(g):
    return deg_sh.at[dst_v.at[(g // IDXB) % 2, g % IDXB]]

  idx_start(0, 0)
  idx_wait(0, 0)
  g_gather_start(0)

  def step(g, _):
    ib = g // IDXB
    qq = g % IDXB
    @pl.when((qq == NB) & (ib + 1 < NBLK))
    def _():
      idx_start(ib + 1, (ib + 1) % 2)
    @pl.when(g + 1 < CPT)
    def _():
      @pl.when(g + 1 >= NB)
      def _():
        g_scatter_wait(g + 1 - NB)
      @pl.when((g + 1) % IDXB == 0)
      def _():
        idx_wait(ib + 1, (ib + 1) % 2)
      g_gather_start(g + 1)
    g_gather_wait(g)
    g_scatter_start(g)
    if with_deg:
      @pl.when(g % 2 == cid)
      def _():
        d = (g // 2) % 2
        @pl.when(g >= 4)
        def _():
          pltpu.make_async_copy(ones_v, g_deg_ref(g - 4), dsem.at[d]).wait()
        pltpu.async_copy(ones_v, g_deg_ref(g), dsem.at[d], add=True)
    return 0
  lax.fori_loop(0, CPT, step, 0)

  for g in range(CPT - NB, CPT):
    g_scatter_wait(g)
  if with_deg:
    for gg in (CPT - 4, CPT - 2):
      g = gg + cid
      pltpu.make_async_copy(ones_v, g_deg_ref(g),
                            dsem.at[(g // 2) % 2]).wait()

  @pl.when(sid < NCH - NS * CPT)
  def _():
    xc = NS * CPT + sid
    pltpu.sync_copy(e_hbm.at[0, pl.ds(xc, 1)], src_v.at[0, pl.ds(0, 1)])
    pltpu.sync_copy(e_hbm.at[1, pl.ds(xc, 1)], dst_v.at[0, pl.ds(0, 1)])
    pltpu.async_copy(h_sh.at[src_v.at[0, 0]], rows_v.at[0],
                     gsem.at[0]).wait()
    pltpu.sync_copy(rows_v.at[0], agg_sh.at[dst_v.at[0, 0]], add=True)
    if with_deg:
      @pl.when(cid == 0)
      def _():
        pltpu.sync_copy(ones_v, deg_sh.at[dst_v.at[0, 0]], add=True)

  plsc.subcore_barrier()

  pltpu.sync_copy(agg_sh.at[pl.ds(row0, RPT)],
                  agg_out.at[pl.ds(row0, RPT), pl.ds(col0, HH)])
  if with_deg:
    pltpu.sync_copy(deg_sh.at[pl.ds(row0, RPT)],
                    deg_out.at[cid, pl.ds(row0, RPT)])


def _make_sc_agg(with_deg):
  mesh = plsc.VectorSubcoreMesh(core_axis_name="c", subcore_axis_name="s")
  out_type = [jax.ShapeDtypeStruct((NP, H), jnp.float32)]
  scratch = [
      pltpu.VMEM_SHARED((NP, HH), jnp.float32),
      pltpu.VMEM_SHARED((NP, HH), jnp.float32),
  ]
  if with_deg:
    out_type.append(jax.ShapeDtypeStruct((NC, NP, DW), jnp.float32))
    scratch.append(pltpu.VMEM_SHARED((NP, DW), jnp.float32))
  nb = 3 if with_deg else 4
  scratch += [
      pltpu.VMEM((2, IDXB, K), jnp.int32),
      pltpu.VMEM((2, IDXB, K), jnp.int32),
      pltpu.VMEM((nb, K, HH), jnp.float32),
      pltpu.VMEM((ZR, HH), jnp.float32),
  ]
  if with_deg:
    scratch += [
        pltpu.VMEM((ZR4, DW), jnp.float32),
        pltpu.VMEM((K, DW), jnp.float32),
    ]
  scratch.append(pltpu.SemaphoreType.DMA((nb,)))
  scratch.append(pltpu.SemaphoreType.DMA((nb,)))
  scratch.append(pltpu.SemaphoreType.DMA((2,)))
  if with_deg:
    scratch.append(pltpu.SemaphoreType.DMA((2,)))
  return pl.kernel(
      functools.partial(_sc_agg_body, with_deg),
      out_type=tuple(out_type),
      mesh=mesh,
      scratch_types=tuple(scratch),
      compiler_params=pltpu.CompilerParams(use_tc_tiling_on_sc=False),
  )


def _pre_body(x_ref, w_ref, b_ref, hf_ref):
  h = jnp.dot(x_ref[...], w_ref[...], preferred_element_type=jnp.float32)
  hf_ref[...] = jnp.maximum(h + b_ref[...], 0.0)


def _conv_body(head, *refs):
  if head:
    (agg_ref, deg_ref, h_ref, llw_ref, llb_ref, lrw_ref, g_ref, b_ref,
     hw_ref, hb_ref, hf_ref, y_ref) = refs
  else:
    (agg_ref, deg_ref, h_ref, llw_ref, llb_ref, lrw_ref, g_ref, b_ref,
     hf_ref) = refs
  deg = deg_ref[0][:, 0:1] + deg_ref[1][:, 0:1]
  inv = 1.0 / jnp.maximum(deg, 1.0)
  z = jnp.dot(agg_ref[...] * inv, llw_ref[...],
              preferred_element_type=jnp.float32)
  z = z + jnp.dot(h_ref[...], lrw_ref[...], preferred_element_type=jnp.float32)
  z = z + llb_ref[...]
  bn_scale = g_ref[...] * (1.0 / jnp.sqrt(1.0 + EPS))
  h = jnp.maximum(z * bn_scale + b_ref[...], 0.0)
  hf_ref[...] = h
  if head:
    y_ref[...] = lax.dot_general(
        hw_ref[...], h, (((1,), (1,)), ((), ())),
        preferred_element_type=jnp.float32) + hb_ref[...]


_row_spec = pl.BlockSpec((ROWBLK, H), lambda i: (i, 0))
_deg_spec = pl.BlockSpec((NC, ROWBLK, DW), lambda i: (0, i, 0))
_full_spec = pl.BlockSpec((H, H), lambda i: (0, 0))
_vec_spec = pl.BlockSpec((1, H), lambda i: (0, 0))


def _tc_pre(x, w, b):
  return pl.pallas_call(
      _pre_body,
      grid=(NP // ROWBLK,),
      in_specs=[_row_spec, _full_spec, _vec_spec],
      out_specs=_row_spec,
      out_shape=jax.ShapeDtypeStruct((NP, H), jnp.float32),
  )(x, w, b)


def _tc_conv(agg, deg, h, llw, llb, lrw, g, b):
  return pl.pallas_call(
      functools.partial(_conv_body, False),
      grid=(NP // ROWBLK,),
      in_specs=[
          _row_spec, _deg_spec,
          _row_spec, _full_spec, _vec_spec, _full_spec, _vec_spec, _vec_spec,
      ],
      out_specs=_row_spec,
      out_shape=jax.ShapeDtypeStruct((NP, H), jnp.float32),
  )(agg, deg, h, llw, llb, lrw, g, b)


def _tc_conv_head(agg, deg, h, llw, llb, lrw, g, b, hw, hb):
  return pl.pallas_call(
      functools.partial(_conv_body, True),
      grid=(NP // ROWBLK,),
      in_specs=[
          _row_spec, _deg_spec,
          _row_spec, _full_spec, _vec_spec, _full_spec, _vec_spec, _vec_spec,
          pl.BlockSpec((1, H), lambda i: (0, 0)),
          pl.BlockSpec((1, 1), lambda i: (0, 0)),
      ],
      out_specs=[_row_spec, pl.BlockSpec((1, ROWBLK), lambda i: (0, i))],
      out_shape=[
          jax.ShapeDtypeStruct((N, H), jnp.float32),
          jax.ShapeDtypeStruct((1, N), jnp.float32),
      ],
  )(agg, deg, h, llw, llb, lrw, g, b, hw, hb)


def kernel(x, edge_index, pre_w, pre_b, c0_ll_w, c0_ll_b, c0_lr_w,
           c1_ll_w, c1_ll_b, c1_lr_w, n0_g, n0_b, n1_g, n1_b,
           head_w, head_b):
  e3 = edge_index.reshape(2, NCH, K)

  h0 = _tc_pre(x, pre_w, pre_b.reshape(1, H))

  agg0, deg = _make_sc_agg(True)(h0, e3)
  h1 = _tc_conv(agg0, deg, h0, c0_ll_w, c0_ll_b.reshape(1, H),
                c0_lr_w, n0_g.reshape(1, H), n0_b.reshape(1, H))

  (agg1,) = _make_sc_agg(False)(h1, e3)
  h2, y = _tc_conv_head(agg1, deg, h1, c1_ll_w, c1_ll_b.reshape(1, H),
                        c1_lr_w, n1_g.reshape(1, H), n1_b.reshape(1, H),
                        head_w.reshape(1, H), head_b.reshape(1, 1))
  return (y[0], h2)


# speedup vs baseline: 11.1533x; 1.0213x over previous
import functools

import jax
import jax.numpy as jnp
from jax import lax
from jax.experimental import pallas as pl
from jax.experimental.pallas import tpu as pltpu
from jax.experimental.pallas import tpu_sc as plsc

N = 10000
E = 320000
D = 128
H = 128
EPS = 1e-5

NC = 2
NS = 16
HH = H // NC
NP = 10240
K = 128
NCH = E // K
CPT = 156
IDXB = 6
NBLK = CPT // IDXB
RPT = NP // NS
ZR = 8
ZR4 = 16
DW = 16

ROWBLK = 2048


def _zero_fill(ref, rows, width):
  z = jnp.zeros((16,), jnp.float32)
  def body(i, _):
    for j in range(width // 16):
      ref[i, pl.ds(j * 16, 16)] = z
    return 0
  lax.fori_loop(0, rows, body, 0)


def _fill_ones(ref, rows):
  o = jnp.ones((16,), jnp.float32)
  def body(i, _):
    ref[i, :] = o
    return 0
  lax.fori_loop(0, rows, body, 0)


def _sc_agg_body(with_deg, *refs):
  if with_deg:
    (h_hbm, e_hbm, agg_out, deg_out,
     h_sh, agg_sh, deg_sh, src_v, dst_v, rows_v, zbuf, zdeg, ones_v,
     gsem, ssem, isem, dsem, zsem) = refs
    NB = 4
  else:
    (h_hbm, e_hbm, agg_out,
     h_sh, agg_sh, src_v, dst_v, rows_v, zbuf,
     gsem, ssem, isem, zsem) = refs
    NB = 4

  cid = lax.axis_index("c")
  sid = lax.axis_index("s")
  row0 = sid * RPT
  col0 = cid * HH
  base = sid * CPT

  nz = 1 + RPT // ZR
  pltpu.async_copy(h_hbm.at[pl.ds(row0, RPT), pl.ds(col0, HH)],
                   h_sh.at[pl.ds(row0, RPT)], zsem)
  _zero_fill(zbuf, ZR, HH)
  for i in range(RPT // ZR):
    pltpu.async_copy(zbuf, agg_sh.at[pl.ds(row0 + i * ZR, ZR)], zsem)
  if with_deg:
    _zero_fill(zdeg, ZR4, DW)
    _fill_ones(ones_v, K)
    for i in range(RPT // ZR4):
      pltpu.async_copy(zdeg, deg_sh.at[pl.ds(row0 + i * ZR4, ZR4)], zsem)
    nz += RPT // ZR4
  pltpu.make_async_copy(h_hbm.at[pl.ds(row0, RPT), pl.ds(col0, HH)],
                        h_sh.at[pl.ds(row0, RPT)], zsem).wait()
  for i in range(RPT // ZR):
    pltpu.make_async_copy(zbuf, agg_sh.at[pl.ds(row0 + i * ZR, ZR)],
                          zsem).wait()
  if with_deg:
    for i in range(RPT // ZR4):
      pltpu.make_async_copy(zdeg, deg_sh.at[pl.ds(row0 + i * ZR4, ZR4)],
                            zsem).wait()

  plsc.subcore_barrier()

  def gather_start(pb, q):
    b = q % NB
    pltpu.async_copy(h_sh.at[src_v.at[pb, q]], rows_v.at[b], gsem.at[b])

  def gather_wait(pb, q):
    b = q % NB
    pltpu.make_async_copy(h_sh.at[src_v.at[pb, q]], rows_v.at[b],
                          gsem.at[b]).wait()

  def scatter_start(pb, q):
    b = q % NB
    pltpu.async_copy(rows_v.at[b], agg_sh.at[dst_v.at[pb, q]], ssem.at[b],
                     add=True)

  def scatter_wait(pb, q):
    b = q % NB
    pltpu.make_async_copy(rows_v.at[b], agg_sh.at[dst_v.at[pb, q]],
                          ssem.at[b]).wait()

  def idx_start(ib, pb):
    pltpu.async_copy(e_hbm.at[0, pl.ds(base + ib * IDXB, IDXB)],
                     src_v.at[pb], isem.at[pb])
    pltpu.async_copy(e_hbm.at[1, pl.ds(base + ib * IDXB, IDXB)],
                     dst_v.at[pb], isem.at[pb])

  def idx_wait(ib, pb):
    pltpu.make_async_copy(e_hbm.at[0, pl.ds(base + ib * IDXB, IDXB)],
                          src_v.at[pb], isem.at[pb]).wait()
    pltpu.make_async_copy(e_hbm.at[1, pl.ds(base + ib * IDXB, IDXB)],
                          dst_v.at[pb], isem.at[pb]).wait()

  def g_gather_start(g):
    gather_start((g // IDXB) % 2, g % IDXB)

  def g_gather_wait(g):
    gather_wait((g // IDXB) % 2, g % IDXB)

  def g_scatter_start(g):
    scatter_start((g // IDXB) % 2, g % IDXB)

  def g_scatter_wait(g):
    scatter_wait((g // IDXB) % 2, g % IDXB)

  def g_deg_ref(g):
    return deg_sh.at[dst_v.at[(g // IDXB) % 2, g % IDXB]]

  idx_start(0, 0)
  idx_wait(0, 0)
  g_gather_start(0)

  def step(g, _):
    ib = g // IDXB
    qq = g % IDXB
    @pl.when((qq == NB - 1) & (ib + 1 < NBLK))
    def _():
      idx_start(ib + 1, (ib + 1) % 2)
    @pl.when(g + 1 < CPT)
    def _():
      @pl.when(g + 1 >= NB)
      def _():
        g_scatter_wait(g + 1 - NB)
      @pl.when((g + 1) % IDXB == 0)
      def _():
        idx_wait(ib + 1, (ib + 1) % 2)
      g_gather_start(g + 1)
    g_gather_wait(g)
    g_scatter_start(g)
    if with_deg:
      @pl.when(g % 2 == cid)
      def _():
        d = (g // 2) % 4
        @pl.when(g >= 8)
        def _():
          pltpu.make_async_copy(ones_v, g_deg_ref(g - 8), dsem.at[d]).wait()
        pltpu.async_copy(ones_v, g_deg_ref(g), dsem.at[d], add=True)
    return 0
  lax.fori_loop(0, CPT, step, 0)

  for g in range(CPT - NB, CPT):
    g_scatter_wait(g)
  if with_deg:
    for gg in (CPT - 8, CPT - 6, CPT - 4, CPT - 2):
      g = gg + cid
      pltpu.make_async_copy(ones_v, g_deg_ref(g),
                            dsem.at[(g // 2) % 4]).wait()

  @pl.when(sid < NCH - NS * CPT)
  def _():
    xc = NS * CPT + sid
    pltpu.sync_copy(e_hbm.at[0, pl.ds(xc, 1)], src_v.at[0, pl.ds(0, 1)])
    pltpu.sync_copy(e_hbm.at[1, pl.ds(xc, 1)], dst_v.at[0, pl.ds(0, 1)])
    pltpu.async_copy(h_sh.at[src_v.at[0, 0]], rows_v.at[0],
                     gsem.at[0]).wait()
    pltpu.sync_copy(rows_v.at[0], agg_sh.at[dst_v.at[0, 0]], add=True)
    if with_deg:
      @pl.when(cid == 0)
      def _():
        pltpu.sync_copy(ones_v, deg_sh.at[dst_v.at[0, 0]], add=True)

  plsc.subcore_barrier()

  pltpu.sync_copy(agg_sh.at[pl.ds(row0, RPT)],
                  agg_out.at[pl.ds(row0, RPT), pl.ds(col0, HH)])
  if with_deg:
    pltpu.sync_copy(deg_sh.at[pl.ds(row0, RPT)],
                    deg_out.at[cid, pl.ds(row0, RPT)])


def _make_sc_agg(with_deg):
  mesh = plsc.VectorSubcoreMesh(core_axis_name="c", subcore_axis_name="s")
  out_type = [jax.ShapeDtypeStruct((NP, H), jnp.float32)]
  scratch = [
      pltpu.VMEM_SHARED((NP, HH), jnp.float32),
      pltpu.VMEM_SHARED((NP, HH), jnp.float32),
  ]
  if with_deg:
    out_type.append(jax.ShapeDtypeStruct((NC, NP, DW), jnp.float32))
    scratch.append(pltpu.VMEM_SHARED((NP, DW), jnp.float32))
  nb = 4
  scratch += [
      pltpu.VMEM((2, IDXB, K), jnp.int32),
      pltpu.VMEM((2, IDXB, K), jnp.int32),
      pltpu.VMEM((nb, K, HH), jnp.float32),
      pltpu.VMEM((ZR, HH), jnp.float32),
  ]
  if with_deg:
    scratch += [
        pltpu.VMEM((ZR4, DW), jnp.float32),
        pltpu.VMEM((K, DW), jnp.float32),
    ]
  scratch.append(pltpu.SemaphoreType.DMA((nb,)))
  scratch.append(pltpu.SemaphoreType.DMA((nb,)))
  scratch.append(pltpu.SemaphoreType.DMA((2,)))
  if with_deg:
    scratch.append(pltpu.SemaphoreType.DMA((4,)))
  scratch.append(pltpu.SemaphoreType.DMA)
  return pl.kernel(
      functools.partial(_sc_agg_body, with_deg),
      out_type=tuple(out_type),
      mesh=mesh,
      scratch_types=tuple(scratch),
      compiler_params=pltpu.CompilerParams(use_tc_tiling_on_sc=False),
  )


def _pre_body(x_ref, w_ref, b_ref, hf_ref):
  h = jnp.dot(x_ref[...], w_ref[...], preferred_element_type=jnp.float32)
  hf_ref[...] = jnp.maximum(h + b_ref[...], 0.0)


def _conv_body(head, *refs):
  if head:
    (agg_ref, deg_ref, h_ref, llw_ref, llb_ref, lrw_ref, g_ref, b_ref,
     hw_ref, hb_ref, hf_ref, y_ref) = refs
  else:
    (agg_ref, deg_ref, h_ref, llw_ref, llb_ref, lrw_ref, g_ref, b_ref,
     hf_ref) = refs
  deg = deg_ref[0][:, 0:1] + deg_ref[1][:, 0:1]
  inv = 1.0 / jnp.maximum(deg, 1.0)
  z = jnp.dot(agg_ref[...] * inv, llw_ref[...],
              preferred_element_type=jnp.float32)
  z = z + jnp.dot(h_ref[...], lrw_ref[...], preferred_element_type=jnp.float32)
  z = z + llb_ref[...]
  bn_scale = g_ref[...] * (1.0 / jnp.sqrt(1.0 + EPS))
  h = jnp.maximum(z * bn_scale + b_ref[...], 0.0)
  hf_ref[...] = h
  if head:
    y_ref[...] = lax.dot_general(
        hw_ref[...], h, (((1,), (1,)), ((), ())),
        preferred_element_type=jnp.float32) + hb_ref[...]


_row_spec = pl.BlockSpec((ROWBLK, H), lambda i: (i, 0))
_deg_spec = pl.BlockSpec((NC, ROWBLK, DW), lambda i: (0, i, 0))
_full_spec = pl.BlockSpec((H, H), lambda i: (0, 0))
_vec_spec = pl.BlockSpec((1, H), lambda i: (0, 0))


def _tc_pre(x, w, b):
  return pl.pallas_call(
      _pre_body,
      grid=(NP // ROWBLK,),
      in_specs=[_row_spec, _full_spec, _vec_spec],
      out_specs=_row_spec,
      out_shape=jax.ShapeDtypeStruct((NP, H), jnp.float32),
  )(x, w, b)


def _tc_conv(agg, deg, h, llw, llb, lrw, g, b):
  return pl.pallas_call(
      functools.partial(_conv_body, False),
      grid=(NP // ROWBLK,),
      in_specs=[
          _row_spec, _deg_spec,
          _row_spec, _full_spec, _vec_spec, _full_spec, _vec_spec, _vec_spec,
      ],
      out_specs=_row_spec,
      out_shape=jax.ShapeDtypeStruct((NP, H), jnp.float32),
  )(agg, deg, h, llw, llb, lrw, g, b)


def _tc_conv_head(agg, deg, h, llw, llb, lrw, g, b, hw, hb):
  return pl.pallas_call(
      functools.partial(_conv_body, True),
      grid=(NP // ROWBLK,),
      in_specs=[
          _row_spec, _deg_spec,
          _row_spec, _full_spec, _vec_spec, _full_spec, _vec_spec, _vec_spec,
          pl.BlockSpec((1, H), lambda i: (0, 0)),
          pl.BlockSpec((1, 1), lambda i: (0, 0)),
      ],
      out_specs=[_row_spec, pl.BlockSpec((1, ROWBLK), lambda i: (0, i))],
      out_shape=[
          jax.ShapeDtypeStruct((N, H), jnp.float32),
          jax.ShapeDtypeStruct((1, N), jnp.float32),
      ],
  )(agg, deg, h, llw, llb, lrw, g, b, hw, hb)


def kernel(x, edge_index, pre_w, pre_b, c0_ll_w, c0_ll_b, c0_lr_w,
           c1_ll_w, c1_ll_b, c1_lr_w, n0_g, n0_b, n1_g, n1_b,
           head_w, head_b):
  e3 = edge_index.reshape(2, NCH, K)

  h0 = _tc_pre(x, pre_w, pre_b.reshape(1, H))

  agg0, deg = _make_sc_agg(True)(h0, e3)
  h1 = _tc_conv(agg0, deg, h0, c0_ll_w, c0_ll_b.reshape(1, H),
                c0_lr_w, n0_g.reshape(1, H), n0_b.reshape(1, H))

  (agg1,) = _make_sc_agg(False)(h1, e3)
  h2, y = _tc_conv_head(agg1, deg, h1, c1_ll_w, c1_ll_b.reshape(1, H),
                        c1_lr_w, n1_g.reshape(1, H), n1_b.reshape(1, H),
                        head_w.reshape(1, H), head_b.reshape(1, 1))
  return (y[0], h2)
